# Initial kernel scaffold; baseline (speedup 1.0000x reference)
import jax
import jax.numpy as jnp
from jax.experimental import pallas as pl


def kernel(x_static, hist_speed, time_idx, day_idx, edge_index, time_emb, day_emb, gru_W_ih, gru_W_hh, gru_b_ih, gru_b_hh, gat1_W, gat1_att_src, gat1_att_dst, gat1_bias, gat2_W, gat2_att_src, gat2_att_dst, gat2_bias):
    raise NotImplementedError("write your pallas kernel here")


# speedup vs baseline: 23.1566x; 23.1566x over previous
import functools

import jax
import jax.numpy as jnp
from jax import lax
from jax.experimental import pallas as pl
from jax.experimental.pallas import tpu as pltpu
from jax.experimental.pallas import tpu_sc as plsc

N = 50000
E = 800000
NPAD = 50176
R = 512
HIST_T = 12
RNN_H = 64
GNN_H = 32
HEADS1 = 4
GNN_IN = 112

NW = 32
W = 128
E_PAD = 802816
EP_TILE = E_PAD // NW
NB = EP_TILE // W
PR = NPAD // 16

_f32 = jnp.float32


def _k1_body(xs_ref, hist_ref, tif_ref, dif_ref,
             temb_ref, demb_ref, wih_ref, whh_ref, bih_ref, bhh_ref,
             w1_ref, a1s_ref, a1d_ref,
             h1_ref, asrc_ref, adst_ref):
    rows = xs_ref.shape[0]
    t_oh = (lax.broadcasted_iota(jnp.int32, (rows, 288), 1) == tif_ref[:, :]).astype(_f32)
    d_oh = (lax.broadcasted_iota(jnp.int32, (rows, 8), 1) == dif_ref[:, :]).astype(_f32)
    t_emb = jnp.dot(t_oh, temb_ref[:, :], preferred_element_type=_f32)
    d_emb = jnp.dot(d_oh, demb_ref[:, :], preferred_element_type=_f32)
    h = jnp.zeros((rows, RNN_H), dtype=_f32)
    wih = wih_ref[:, :]
    bih = bih_ref[:, :]
    bhh = bhh_ref[:, :]
    for t in range(HIST_T):
        x_t = hist_ref[:, t:t + 1]
        gi = x_t * wih + bih
        gh = jnp.dot(h, whh_ref[:, :], preferred_element_type=_f32) + bhh
        i_r = gi[:, 0:RNN_H]
        i_z = gi[:, RNN_H:2 * RNN_H]
        i_n = gi[:, 2 * RNN_H:3 * RNN_H]
        h_r = gh[:, 0:RNN_H]
        h_z = gh[:, RNN_H:2 * RNN_H]
        h_n = gh[:, 2 * RNN_H:3 * RNN_H]
        r = jax.nn.sigmoid(i_r + h_r)
        z = jax.nn.sigmoid(i_z + h_z)
        nn_ = jnp.tanh(i_n + r * h_n)
        h = (1.0 - z) * nn_ + z * h
    feat = jnp.concatenate([xs_ref[:, :], t_emb, d_emb, h], axis=1)
    h1 = jnp.dot(feat, w1_ref[:, :], preferred_element_type=_f32)
    h1_ref[:, :] = h1
    zs = jnp.zeros((rows, 12), dtype=_f32)
    a_s = []
    a_d = []
    for hd in range(HEADS1):
        blk = h1[:, hd * GNN_H:(hd + 1) * GNN_H]
        a_s.append(jnp.sum(blk * a1s_ref[:, hd * GNN_H:(hd + 1) * GNN_H],
                           axis=1, keepdims=True))
        a_d.append(jnp.sum(blk * a1d_ref[:, hd * GNN_H:(hd + 1) * GNN_H],
                           axis=1, keepdims=True))
    asrc_ref[:, :] = jnp.concatenate(a_s + [zs], axis=1)
    adst_ref[:, :] = jnp.concatenate(a_d + [zs], axis=1)


def _run_k1(xs, hist, tif, dif, temb, demb, wih, whh, bih, bhh, w1, a1s, a1d):
    nblk = NPAD // R
    full = lambda shape: pl.BlockSpec(shape, lambda i: (0, 0))
    rowblk = lambda c: pl.BlockSpec((R, c), lambda i: (i, 0))
    return pl.pallas_call(
        _k1_body,
        grid=(nblk,),
        in_specs=[rowblk(16), rowblk(HIST_T), rowblk(1), rowblk(1),
                  full((288, 16)), full((8, 16)), full((1, 192)),
                  full((RNN_H, 192)), full((1, 192)), full((1, 192)),
                  full((GNN_IN, 128)), full((1, 128)), full((1, 128))],
        out_specs=[rowblk(128), rowblk(16), rowblk(16)],
        out_shape=[jax.ShapeDtypeStruct((NPAD, 128), _f32),
                   jax.ShapeDtypeStruct((NPAD, 16), _f32),
                   jax.ShapeDtypeStruct((NPAD, 16), _f32)],
    )(xs, hist, tif, dif, temb, demb, wih, whh, bih, bhh, w1, a1s, a1d)


def _mid_body(heads, scale, d0_ref, d1_ref, asrc_ref, adst_ref, h_ref,
              bdst_ref, selfm_ref):
    rows = d0_ref.shape[0]
    a_s = asrc_ref[:, 0:4]
    a_d = adst_ref[:, 0:4]
    t = a_s + a_d
    ex_self = jnp.exp(jnp.where(t > 0, t, 0.2 * t))
    denom = d0_ref[:, 0:4] + d1_ref[:, 0:4] + ex_self
    rdenom = scale / (denom + 1e-16)
    zs8 = jnp.zeros((rows, 8), dtype=_f32)
    bdst_ref[:, :] = jnp.concatenate([a_d, rdenom, zs8], axis=1)
    alpha_self = ex_self * rdenom
    acc = jnp.zeros((rows, GNN_H), dtype=_f32)
    for hd in range(heads):
        acc = acc + alpha_self[:, hd:hd + 1] * h_ref[:, hd * GNN_H:(hd + 1) * GNN_H]
    selfm_ref[:, :] = acc


def _run_mid(heads, scale, d0, d1, asrc, adst, h):
    nblk = NPAD // R
    hw = h.shape[1]
    rowblk = lambda c: pl.BlockSpec((R, c), lambda i: (i, 0))
    return pl.pallas_call(
        functools.partial(_mid_body, heads, scale),
        grid=(nblk,),
        in_specs=[rowblk(16), rowblk(16), rowblk(16), rowblk(16), rowblk(hw)],
        out_specs=[rowblk(16), rowblk(GNN_H)],
        out_shape=[jax.ShapeDtypeStruct((NPAD, 16), _f32),
                   jax.ShapeDtypeStruct((NPAD, GNN_H), _f32)],
    )(d0, d1, asrc, adst, h)


def _k3_body(p0_ref, p1_ref, selfm_ref, b1_ref, w2_ref, a2s_ref, a2d_ref,
             h2_ref, asrc_ref, adst_ref):
    rows = p0_ref.shape[0]
    x = p0_ref[:, :] + p1_ref[:, :] + selfm_ref[:, :] + b1_ref[:, :]
    x = jnp.maximum(x, 0.0)
    h2 = jnp.dot(x, w2_ref[:, :], preferred_element_type=_f32)
    h2_ref[:, :] = h2
    zs = jnp.zeros((rows, 15), dtype=_f32)
    a_s = jnp.sum(h2 * a2s_ref[:, :], axis=1, keepdims=True)
    a_d = jnp.sum(h2 * a2d_ref[:, :], axis=1, keepdims=True)
    asrc_ref[:, :] = jnp.concatenate([a_s, zs], axis=1)
    adst_ref[:, :] = jnp.concatenate([a_d, zs], axis=1)


def _run_k3(p0, p1, selfm, b1, w2, a2s, a2d):
    nblk = NPAD // R
    full = lambda shape: pl.BlockSpec(shape, lambda i: (0, 0))
    rowblk = lambda c: pl.BlockSpec((R, c), lambda i: (i, 0))
    return pl.pallas_call(
        _k3_body,
        grid=(nblk,),
        in_specs=[rowblk(GNN_H), rowblk(GNN_H), rowblk(GNN_H),
                  full((1, GNN_H)), full((GNN_H, GNN_H)),
                  full((1, GNN_H)), full((1, GNN_H))],
        out_specs=[rowblk(GNN_H), rowblk(16), rowblk(16)],
        out_shape=[jax.ShapeDtypeStruct((NPAD, GNN_H), _f32),
                   jax.ShapeDtypeStruct((NPAD, 16), _f32),
                   jax.ShapeDtypeStruct((NPAD, 16), _f32)],
    )(p0, p1, selfm, b1, w2, a2s, a2d)


def _k5_body(p0_ref, p1_ref, selfm_ref, b2_ref, out_ref):
    x = p0_ref[:, :] + p1_ref[:, :] + selfm_ref[:, :] + b2_ref[:, :]
    out_ref[:, :] = jnp.maximum(x, 0.0)


def _run_k5(p0, p1, selfm, b2):
    nblk = NPAD // R
    full = lambda shape: pl.BlockSpec(shape, lambda i: (0, 0))
    rowblk = lambda c: pl.BlockSpec((R, c), lambda i: (i, 0))
    return pl.pallas_call(
        _k5_body,
        grid=(nblk,),
        in_specs=[rowblk(GNN_H), rowblk(GNN_H), rowblk(GNN_H), full((1, GNN_H))],
        out_specs=rowblk(GNN_H),
        out_shape=jax.ShapeDtypeStruct((NPAD, GNN_H), _f32),
    )(p0, p1, selfm, b2)


def _make_sc_pass_a():
    mesh = plsc.VectorSubcoreMesh(core_axis_name="c", subcore_axis_name="s")

    @functools.partial(
        pl.kernel, mesh=mesh,
        compiler_params=pltpu.CompilerParams(use_tc_tiling_on_sc=False),
        out_type=jax.ShapeDtypeStruct((2, NPAD, 16), _f32),
        scratch_types=[
            pltpu.VMEM((W,), jnp.int32),
            pltpu.VMEM((W,), jnp.int32),
            pltpu.VMEM((W, 16), _f32),
            pltpu.VMEM((W, 16), _f32),
            pltpu.VMEM((W, 16), _f32),
            pltpu.VMEM((64, 16), _f32),
            pltpu.VMEM_SHARED((NPAD, 16), _f32),
        ],
    )
    def ka(src_hbm, dst_hbm, tabs_hbm, tabd_hbm, out_hbm,
           sidx, didx, srows, drows, exr, zbuf, acc):
        cid = lax.axis_index("c")
        sid = lax.axis_index("s")
        wid = sid * 2 + cid

        @pl.loop(0, 64)
        def _(i):
            zbuf[i, :] = jnp.zeros((16,), _f32)

        @pl.loop(0, PR // 64)
        def _(j):
            pltpu.sync_copy(zbuf, acc.at[pl.ds(sid * PR + j * 64, 64)])

        plsc.subcore_barrier()
        base = wid * EP_TILE

        @pl.loop(0, NB)
        def _(j):
            off = base + j * W
            pltpu.sync_copy(src_hbm.at[pl.ds(off, W)], sidx)
            pltpu.sync_copy(dst_hbm.at[pl.ds(off, W)], didx)
            pltpu.sync_copy(tabs_hbm.at[sidx], srows)
            pltpu.sync_copy(tabd_hbm.at[didx], drows)

            @pl.loop(0, W)
            def _(e):
                t = srows.at[e][...] + drows.at[e][...]
                t = jnp.where(t > 0, t, 0.2 * t)
                exr.at[e][...] = jnp.exp(t)

            pltpu.sync_copy(exr, acc.at[didx], add=True)

        plsc.subcore_barrier()
        pltpu.sync_copy(acc.at[pl.ds(sid * PR, PR)],
                        out_hbm.at[cid, pl.ds(sid * PR, PR)])

    return ka


def _make_sc_pass_b(hw, heads):
    mesh = plsc.VectorSubcoreMesh(core_axis_name="c", subcore_axis_name="s")

    @functools.partial(
        pl.kernel, mesh=mesh,
        compiler_params=pltpu.CompilerParams(use_tc_tiling_on_sc=False),
        out_type=jax.ShapeDtypeStruct((2, NPAD, GNN_H), _f32),
        scratch_types=[
            pltpu.VMEM((W,), jnp.int32),
            pltpu.VMEM((W,), jnp.int32),
            pltpu.VMEM((W, 16), _f32),
            pltpu.VMEM((W, 16), _f32),
            pltpu.VMEM((W, hw), _f32),
            pltpu.VMEM((W, GNN_H), _f32),
            pltpu.VMEM((64, GNN_H), _f32),
            pltpu.VMEM_SHARED((NPAD, GNN_H), _f32),
        ],
    )
    def kb(src_hbm, dst_hbm, tabs_hbm, tabd_hbm, h_hbm, out_hbm,
           sidx, didx, srows, brows, hrows, mrows, zbuf, acc):
        cid = lax.axis_index("c")
        sid = lax.axis_index("s")
        wid = sid * 2 + cid

        @pl.loop(0, 64)
        def _(i):
            zbuf[i, pl.ds(0, 16)] = jnp.zeros((16,), _f32)
            zbuf[i, pl.ds(16, 16)] = jnp.zeros((16,), _f32)

        @pl.loop(0, PR // 64)
        def _(j):
            pltpu.sync_copy(zbuf, acc.at[pl.ds(sid * PR + j * 64, 64)])

        plsc.subcore_barrier()
        base = wid * EP_TILE
        lane = lax.iota(jnp.int32, 16)
        exmask = lane < 4

        @pl.loop(0, NB)
        def _(j):
            off = base + j * W
            pltpu.sync_copy(src_hbm.at[pl.ds(off, W)], sidx)
            pltpu.sync_copy(dst_hbm.at[pl.ds(off, W)], didx)
            pltpu.sync_copy(tabs_hbm.at[sidx], srows)
            pltpu.sync_copy(tabd_hbm.at[didx], brows)
            pltpu.sync_copy(h_hbm.at[sidx], hrows)

            @pl.loop(0, W)
            def _(e):
                t = srows.at[e][...] + brows.at[e][...]
                t = jnp.where(t > 0, t, 0.2 * t)
                v = jnp.where(exmask, jnp.exp(t), t)
                hrow = hrows.at[e]
                m0 = jnp.zeros((16,), _f32)
                m1 = jnp.zeros((16,), _f32)
                for hd in range(heads):
                    a_sc = v[hd] * v[4 + hd]
                    ab = lax.broadcast_in_dim(a_sc, (16,), ())
                    m0 = m0 + ab * hrow[pl.ds(hd * GNN_H, 16)]
                    m1 = m1 + ab * hrow[pl.ds(hd * GNN_H + 16, 16)]
                mrow = mrows.at[e]
                mrow[pl.ds(0, 16)] = m0
                mrow[pl.ds(16, 16)] = m1

            pltpu.sync_copy(mrows, acc.at[didx], add=True)

        plsc.subcore_barrier()
        pltpu.sync_copy(acc.at[pl.ds(sid * PR, PR)],
                        out_hbm.at[cid, pl.ds(sid * PR, PR)])

    return kb


_sc_pass_a = _make_sc_pass_a()
_sc_pass_b1 = _make_sc_pass_b(128, HEADS1)
_sc_pass_b2 = _make_sc_pass_b(GNN_H, 1)


def kernel(x_static, hist_speed, time_idx, day_idx, edge_index, time_emb,
           day_emb, gru_W_ih, gru_W_hh, gru_b_ih, gru_b_hh, gat1_W,
           gat1_att_src, gat1_att_dst, gat1_bias, gat2_W, gat2_att_src,
           gat2_att_dst, gat2_bias):
    pad_n = NPAD - N
    xs = jnp.pad(x_static, ((0, pad_n), (0, 0)))
    hist = jnp.pad(hist_speed.reshape(N, HIST_T), ((0, pad_n), (0, 0)))
    tif = jnp.pad(time_idx.astype(jnp.int32).reshape(N, 1), ((0, pad_n), (0, 0)))
    dif = jnp.pad(day_idx.astype(jnp.int32).reshape(N, 1), ((0, pad_n), (0, 0)))
    demb = jnp.pad(day_emb, ((0, 1), (0, 0)))
    wih = gru_W_ih.reshape(1, 3 * RNN_H)
    whh = gru_W_hh.T
    bih = gru_b_ih.reshape(1, 3 * RNN_H)
    bhh = gru_b_hh.reshape(1, 3 * RNN_H)
    a1s = gat1_att_src.reshape(1, HEADS1 * GNN_H)
    a1d = gat1_att_dst.reshape(1, HEADS1 * GNN_H)
    a2s = gat2_att_src.reshape(1, GNN_H)
    a2d = gat2_att_dst.reshape(1, GNN_H)
    b1 = gat1_bias.reshape(1, GNN_H)
    b2 = gat2_bias.reshape(1, GNN_H)

    pad_e = E_PAD - E
    fill = jnp.full((pad_e,), NPAD - 1, jnp.int32)
    src = jnp.concatenate([edge_index[0].astype(jnp.int32), fill])
    dst = jnp.concatenate([edge_index[1].astype(jnp.int32), fill])

    h1, asrc1, adst1 = _run_k1(xs, hist, tif, dif, time_emb, demb, wih, whh,
                               bih, bhh, gat1_W, a1s, a1d)
    dpart1 = _sc_pass_a(src, dst, asrc1, adst1)
    bdst1, selfm1 = _run_mid(HEADS1, 0.25, dpart1[0], dpart1[1],
                             asrc1, adst1, h1)
    opart1 = _sc_pass_b1(src, dst, asrc1, bdst1, h1)
    h2, asrc2, adst2 = _run_k3(opart1[0], opart1[1], selfm1, b1,
                               gat2_W, a2s, a2d)
    dpart2 = _sc_pass_a(src, dst, asrc2, adst2)
    bdst2, selfm2 = _run_mid(1, 1.0, dpart2[0], dpart2[1], asrc2, adst2, h2)
    opart2 = _sc_pass_b2(src, dst, asrc2, bdst2, h2)
    out = _run_k5(opart2[0], opart2[1], selfm2, b2)
    return out[:N]


# speedup vs baseline: 38.6884x; 1.6707x over previous
import functools

import jax
import jax.numpy as jnp
from jax import lax
from jax.experimental import pallas as pl
from jax.experimental.pallas import tpu as pltpu
from jax.experimental.pallas import tpu_sc as plsc

N = 50000
E = 800000
NPAD = 50176
R = 512
HIST_T = 12
RNN_H = 64
GNN_H = 32
HEADS1 = 4
GNN_IN = 112

NW = 32
W = 128
E_PAD = 802816
EP_TILE = E_PAD // NW
NB = EP_TILE // W
PR = NPAD // 16

_f32 = jnp.float32


def _k1_body(xs_ref, hist_ref, tif_ref, dif_ref,
             temb_ref, demb_ref, wih_ref, whh_ref, bih_ref, bhh_ref,
             w1_ref, a1s_ref, a1d_ref,
             h1_ref, asrc_ref, adst_ref):
    rows = xs_ref.shape[0]
    t_oh = (lax.broadcasted_iota(jnp.int32, (rows, 288), 1) == tif_ref[:, :]).astype(_f32)
    d_oh = (lax.broadcasted_iota(jnp.int32, (rows, 8), 1) == dif_ref[:, :]).astype(_f32)
    t_emb = jnp.dot(t_oh, temb_ref[:, :], preferred_element_type=_f32)
    d_emb = jnp.dot(d_oh, demb_ref[:, :], preferred_element_type=_f32)
    h = jnp.zeros((rows, RNN_H), dtype=_f32)
    wih = wih_ref[:, :]
    bih = bih_ref[:, :]
    bhh = bhh_ref[:, :]
    for t in range(HIST_T):
        x_t = hist_ref[:, t:t + 1]
        gi = x_t * wih + bih
        gh = jnp.dot(h, whh_ref[:, :], preferred_element_type=_f32) + bhh
        i_r = gi[:, 0:RNN_H]
        i_z = gi[:, RNN_H:2 * RNN_H]
        i_n = gi[:, 2 * RNN_H:3 * RNN_H]
        h_r = gh[:, 0:RNN_H]
        h_z = gh[:, RNN_H:2 * RNN_H]
        h_n = gh[:, 2 * RNN_H:3 * RNN_H]
        r = jax.nn.sigmoid(i_r + h_r)
        z = jax.nn.sigmoid(i_z + h_z)
        nn_ = jnp.tanh(i_n + r * h_n)
        h = (1.0 - z) * nn_ + z * h
    feat = jnp.concatenate([xs_ref[:, :], t_emb, d_emb, h], axis=1)
    h1 = jnp.dot(feat, w1_ref[:, :], preferred_element_type=_f32)
    h1_ref[:, :] = h1
    zs = jnp.zeros((rows, 12), dtype=_f32)
    a_s = []
    a_d = []
    for hd in range(HEADS1):
        blk = h1[:, hd * GNN_H:(hd + 1) * GNN_H]
        a_s.append(jnp.sum(blk * a1s_ref[:, hd * GNN_H:(hd + 1) * GNN_H],
                           axis=1, keepdims=True))
        a_d.append(jnp.sum(blk * a1d_ref[:, hd * GNN_H:(hd + 1) * GNN_H],
                           axis=1, keepdims=True))
    asrc_ref[:, :] = jnp.concatenate(a_s + [zs], axis=1)
    adst_ref[:, :] = jnp.concatenate(a_d + [zs], axis=1)


def _run_k1(xs, hist, tif, dif, temb, demb, wih, whh, bih, bhh, w1, a1s, a1d):
    nblk = NPAD // R
    full = lambda shape: pl.BlockSpec(shape, lambda i: (0, 0))
    rowblk = lambda c: pl.BlockSpec((R, c), lambda i: (i, 0))
    return pl.pallas_call(
        _k1_body,
        grid=(nblk,),
        in_specs=[rowblk(16), rowblk(HIST_T), rowblk(1), rowblk(1),
                  full((288, 16)), full((8, 16)), full((1, 192)),
                  full((RNN_H, 192)), full((1, 192)), full((1, 192)),
                  full((GNN_IN, 128)), full((1, 128)), full((1, 128))],
        out_specs=[rowblk(128), rowblk(16), rowblk(16)],
        out_shape=[jax.ShapeDtypeStruct((NPAD, 128), _f32),
                   jax.ShapeDtypeStruct((NPAD, 16), _f32),
                   jax.ShapeDtypeStruct((NPAD, 16), _f32)],
    )(xs, hist, tif, dif, temb, demb, wih, whh, bih, bhh, w1, a1s, a1d)


def _mid_body(heads, scale, d0_ref, d1_ref, asrc_ref, adst_ref, h_ref,
              bdst_ref, selfm_ref):
    rows = d0_ref.shape[0]
    a_s = asrc_ref[:, 0:4]
    a_d = adst_ref[:, 0:4]
    t = a_s + a_d
    ex_self = jnp.exp(jnp.where(t > 0, t, 0.2 * t))
    denom = d0_ref[:, 0:4] + d1_ref[:, 0:4] + ex_self
    rdenom = scale / (denom + 1e-16)
    zs8 = jnp.zeros((rows, 8), dtype=_f32)
    bdst_ref[:, :] = jnp.concatenate([a_d, rdenom, zs8], axis=1)
    alpha_self = ex_self * rdenom
    acc = jnp.zeros((rows, GNN_H), dtype=_f32)
    for hd in range(heads):
        acc = acc + alpha_self[:, hd:hd + 1] * h_ref[:, hd * GNN_H:(hd + 1) * GNN_H]
    selfm_ref[:, :] = acc


def _run_mid(heads, scale, d0, d1, asrc, adst, h):
    nblk = NPAD // R
    hw = h.shape[1]
    rowblk = lambda c: pl.BlockSpec((R, c), lambda i: (i, 0))
    return pl.pallas_call(
        functools.partial(_mid_body, heads, scale),
        grid=(nblk,),
        in_specs=[rowblk(16), rowblk(16), rowblk(16), rowblk(16), rowblk(hw)],
        out_specs=[rowblk(16), rowblk(GNN_H)],
        out_shape=[jax.ShapeDtypeStruct((NPAD, 16), _f32),
                   jax.ShapeDtypeStruct((NPAD, GNN_H), _f32)],
    )(d0, d1, asrc, adst, h)


def _k3_body(p0_ref, p1_ref, selfm_ref, b1_ref, w2_ref, a2s_ref, a2d_ref,
             h2_ref, asrc_ref, adst_ref):
    rows = p0_ref.shape[0]
    x = p0_ref[:, :] + p1_ref[:, :] + selfm_ref[:, :] + b1_ref[:, :]
    x = jnp.maximum(x, 0.0)
    h2 = jnp.dot(x, w2_ref[:, :], preferred_element_type=_f32)
    h2_ref[:, :] = h2
    zs = jnp.zeros((rows, 15), dtype=_f32)
    a_s = jnp.sum(h2 * a2s_ref[:, :], axis=1, keepdims=True)
    a_d = jnp.sum(h2 * a2d_ref[:, :], axis=1, keepdims=True)
    asrc_ref[:, :] = jnp.concatenate([a_s, zs], axis=1)
    adst_ref[:, :] = jnp.concatenate([a_d, zs], axis=1)


def _run_k3(p0, p1, selfm, b1, w2, a2s, a2d):
    nblk = NPAD // R
    full = lambda shape: pl.BlockSpec(shape, lambda i: (0, 0))
    rowblk = lambda c: pl.BlockSpec((R, c), lambda i: (i, 0))
    return pl.pallas_call(
        _k3_body,
        grid=(nblk,),
        in_specs=[rowblk(GNN_H), rowblk(GNN_H), rowblk(GNN_H),
                  full((1, GNN_H)), full((GNN_H, GNN_H)),
                  full((1, GNN_H)), full((1, GNN_H))],
        out_specs=[rowblk(GNN_H), rowblk(16), rowblk(16)],
        out_shape=[jax.ShapeDtypeStruct((NPAD, GNN_H), _f32),
                   jax.ShapeDtypeStruct((NPAD, 16), _f32),
                   jax.ShapeDtypeStruct((NPAD, 16), _f32)],
    )(p0, p1, selfm, b1, w2, a2s, a2d)


def _k5_body(p0_ref, p1_ref, selfm_ref, b2_ref, out_ref):
    x = p0_ref[:, :] + p1_ref[:, :] + selfm_ref[:, :] + b2_ref[:, :]
    out_ref[:, :] = jnp.maximum(x, 0.0)


def _run_k5(p0, p1, selfm, b2):
    nblk = NPAD // R
    full = lambda shape: pl.BlockSpec(shape, lambda i: (0, 0))
    rowblk = lambda c: pl.BlockSpec((R, c), lambda i: (i, 0))
    return pl.pallas_call(
        _k5_body,
        grid=(nblk,),
        in_specs=[rowblk(GNN_H), rowblk(GNN_H), rowblk(GNN_H), full((1, GNN_H))],
        out_specs=rowblk(GNN_H),
        out_shape=jax.ShapeDtypeStruct((NPAD, GNN_H), _f32),
    )(p0, p1, selfm, b2)


def _make_sc_pass(mode, hw=0, heads=0, w=W):
    nb = EP_TILE // w
    ow = 16 if mode == "a" else GNN_H
    mesh = plsc.VectorSubcoreMesh(core_axis_name="c", subcore_axis_name="s")
    scratch = (
        [pltpu.VMEM((w,), jnp.int32)] * 6 +
        [pltpu.VMEM((w, 16), _f32)] * 4 +
        ([pltpu.VMEM((w, hw), _f32)] * 2 if mode == "b" else []) +
        [pltpu.VMEM((w, ow), _f32)] * 2 +
        [pltpu.VMEM((64, ow), _f32)] +
        [pltpu.VMEM_SHARED((NPAD, ow), _f32)] +
        [pltpu.SemaphoreType.DMA] * 6
    )

    @functools.partial(
        pl.kernel, mesh=mesh,
        compiler_params=pltpu.CompilerParams(use_tc_tiling_on_sc=False),
        out_type=jax.ShapeDtypeStruct((2, NPAD, ow), _f32),
        scratch_types=scratch,
    )
    def kp(*refs):
        if mode == "a":
            (src_hbm, dst_hbm, tabs_hbm, tabd_hbm, out_hbm,
             sidx0, didx0, scidx0, sidx1, didx1, scidx1,
             srows0, drows0, srows1, drows1,
             orows0, orows1, zbuf, acc,
             gsem0, gsem1, idxsem0, idxsem1, scsem0, scsem1) = refs
            hrows = None
        else:
            (src_hbm, dst_hbm, tabs_hbm, tabd_hbm, h_hbm, out_hbm,
             sidx0, didx0, scidx0, sidx1, didx1, scidx1,
             srows0, drows0, srows1, drows1, hrows0, hrows1,
             orows0, orows1, zbuf, acc,
             gsem0, gsem1, idxsem0, idxsem1, scsem0, scsem1) = refs
            hrows = (hrows0, hrows1)
        sidx = (sidx0, sidx1)
        didx = (didx0, didx1)
        scidx = (scidx0, scidx1)
        srows = (srows0, srows1)
        drows = (drows0, drows1)
        orows = (orows0, orows1)
        gsem = (gsem0, gsem1)
        idxsem = (idxsem0, idxsem1)
        scsem = (scsem0, scsem1)

        cid = lax.axis_index("c")
        sid = lax.axis_index("s")
        wid = sid * 2 + cid
        base = wid * EP_TILE
        lane = lax.iota(jnp.int32, 16)
        exmask = lane < 4

        @pl.loop(0, 64)
        def _(i):
            for c in range(ow // 16):
                zbuf[i, pl.ds(c * 16, 16)] = jnp.zeros((16,), _f32)

        @pl.loop(0, PR // 64)
        def _(j):
            pltpu.sync_copy(zbuf, acc.at[pl.ds(sid * PR + j * 64, 64)])

        plsc.subcore_barrier()

        def issue_idx(wi, b):
            off = base + wi * w
            pltpu.async_copy(src_hbm.at[pl.ds(off, w)], sidx[b], idxsem[b])
            pltpu.async_copy(dst_hbm.at[pl.ds(off, w)], didx[b], idxsem[b])

        def drain_idx(b):
            pltpu.make_async_copy(src_hbm.at[pl.ds(0, w)], sidx[b],
                                  idxsem[b]).wait()
            pltpu.make_async_copy(dst_hbm.at[pl.ds(0, w)], didx[b],
                                  idxsem[b]).wait()

        def issue_gathers(b):
            pltpu.async_copy(tabs_hbm.at[sidx[b]], srows[b], gsem[b])
            pltpu.async_copy(tabd_hbm.at[didx[b]], drows[b], gsem[b])
            if mode == "b":
                pltpu.async_copy(h_hbm.at[sidx[b]], hrows[b], gsem[b])

        def drain_gathers(b):
            pltpu.make_async_copy(tabs_hbm.at[pl.ds(0, w)], srows[b],
                                  gsem[b]).wait()
            pltpu.make_async_copy(tabd_hbm.at[pl.ds(0, w)], drows[b],
                                  gsem[b]).wait()
            if mode == "b":
                pltpu.make_async_copy(h_hbm.at[pl.ds(0, w)], hrows[b],
                                      gsem[b]).wait()

        def drain_scatter(b):
            pltpu.make_async_copy(out_hbm.at[0, pl.ds(0, w)], orows[b],
                                  scsem[b]).wait()

        def compute(b):
            @pl.loop(0, w)
            def _(e):
                t = srows[b].at[e][...] + drows[b].at[e][...]
                t = jnp.where(t > 0, t, 0.2 * t)
                if mode == "a":
                    orows[b].at[e][...] = jnp.exp(t)
                else:
                    v = jnp.where(exmask, jnp.exp(t), t)
                    hrow = hrows[b].at[e]
                    m0 = jnp.zeros((16,), _f32)
                    m1 = jnp.zeros((16,), _f32)
                    for hd in range(heads):
                        a_sc = v[hd] * v[4 + hd]
                        ab = lax.broadcast_in_dim(a_sc, (16,), ())
                        m0 = m0 + ab * hrow[pl.ds(hd * GNN_H, 16)]
                        m1 = m1 + ab * hrow[pl.ds(hd * GNN_H + 16, 16)]
                    orow = orows[b].at[e]
                    orow[pl.ds(0, 16)] = m0
                    orow[pl.ds(16, 16)] = m1

        pltpu.sync_copy(src_hbm.at[pl.ds(base, w)], sidx[0])
        pltpu.sync_copy(dst_hbm.at[pl.ds(base, w)], didx[0])
        issue_gathers(0)
        issue_idx(1, 1)

        @pl.loop(0, nb // 2)
        def _(tt):
            for b in range(2):
                o = 1 - b
                wi = tt * 2 + b

                @pl.when(wi + 1 < nb)
                def _():
                    drain_idx(o)
                    issue_gathers(o)

                drain_gathers(b)

                @pl.when(wi >= 2)
                def _():
                    drain_scatter(b)

                for k in range(w // 16):
                    scidx[b][pl.ds(k * 16, 16)] = didx[b][pl.ds(k * 16, 16)]

                @pl.when(wi + 2 < nb)
                def _():
                    issue_idx(wi + 2, b)

                compute(b)
                pltpu.async_copy(orows[b], acc.at[scidx[b]], scsem[b],
                                 add=True)

        drain_scatter(0)
        drain_scatter(1)
        plsc.subcore_barrier()
        pltpu.sync_copy(acc.at[pl.ds(sid * PR, PR)],
                        out_hbm.at[cid, pl.ds(sid * PR, PR)])

    return kp


_sc_pass_a = _make_sc_pass("a")
_sc_pass_b1 = _make_sc_pass("b", 128, HEADS1, w=64)
_sc_pass_b2 = _make_sc_pass("b", GNN_H, 1)


def kernel(x_static, hist_speed, time_idx, day_idx, edge_index, time_emb,
           day_emb, gru_W_ih, gru_W_hh, gru_b_ih, gru_b_hh, gat1_W,
           gat1_att_src, gat1_att_dst, gat1_bias, gat2_W, gat2_att_src,
           gat2_att_dst, gat2_bias):
    pad_n = NPAD - N
    xs = jnp.pad(x_static, ((0, pad_n), (0, 0)))
    hist = jnp.pad(hist_speed.reshape(N, HIST_T), ((0, pad_n), (0, 0)))
    tif = jnp.pad(time_idx.astype(jnp.int32).reshape(N, 1), ((0, pad_n), (0, 0)))
    dif = jnp.pad(day_idx.astype(jnp.int32).reshape(N, 1), ((0, pad_n), (0, 0)))
    demb = jnp.pad(day_emb, ((0, 1), (0, 0)))
    wih = gru_W_ih.reshape(1, 3 * RNN_H)
    whh = gru_W_hh.T
    bih = gru_b_ih.reshape(1, 3 * RNN_H)
    bhh = gru_b_hh.reshape(1, 3 * RNN_H)
    a1s = gat1_att_src.reshape(1, HEADS1 * GNN_H)
    a1d = gat1_att_dst.reshape(1, HEADS1 * GNN_H)
    a2s = gat2_att_src.reshape(1, GNN_H)
    a2d = gat2_att_dst.reshape(1, GNN_H)
    b1 = gat1_bias.reshape(1, GNN_H)
    b2 = gat2_bias.reshape(1, GNN_H)

    pad_e = E_PAD - E
    fill = jnp.full((pad_e,), NPAD - 1, jnp.int32)
    src = jnp.concatenate([edge_index[0].astype(jnp.int32), fill])
    dst = jnp.concatenate([edge_index[1].astype(jnp.int32), fill])

    h1, asrc1, adst1 = _run_k1(xs, hist, tif, dif, time_emb, demb, wih, whh,
                               bih, bhh, gat1_W, a1s, a1d)
    dpart1 = _sc_pass_a(src, dst, asrc1, adst1)
    bdst1, selfm1 = _run_mid(HEADS1, 0.25, dpart1[0], dpart1[1],
                             asrc1, adst1, h1)
    opart1 = _sc_pass_b1(src, dst, asrc1, bdst1, h1)
    h2, asrc2, adst2 = _run_k3(opart1[0], opart1[1], selfm1, b1,
                               gat2_W, a2s, a2d)
    dpart2 = _sc_pass_a(src, dst, asrc2, adst2)
    bdst2, selfm2 = _run_mid(1, 1.0, dpart2[0], dpart2[1], asrc2, adst2, h2)
    opart2 = _sc_pass_b2(src, dst, asrc2, bdst2, h2)
    out = _run_k5(opart2[0], opart2[1], selfm2, b2)
    return out[:N]


# speedup vs baseline: 62.7805x; 1.6227x over previous
import functools

import jax
import jax.numpy as jnp
from jax import lax
from jax.experimental import pallas as pl
from jax.experimental.pallas import tpu as pltpu
from jax.experimental.pallas import tpu_sc as plsc

N = 50000
E = 800000
NPAD = 50176
R = 1024
HIST_T = 12
RNN_H = 64
GNN_H = 32
HEADS1 = 4
GNN_IN = 112

NW = 32
W = 128
E_PAD = 802816
EP_TILE = E_PAD // NW
NB = EP_TILE // W
PR = NPAD // 16

_f32 = jnp.float32


def _k1_body(xs_ref, hist_ref, tif_ref, dif_ref,
             temb_ref, demb_ref, wih_ref, whh_ref, bih_ref, bhh_ref,
             w1_ref, asm_ref, adm_ref,
             h1_ref, asrc_ref, adst_ref):
    rows = xs_ref.shape[0]
    t_oh = (lax.broadcasted_iota(jnp.int32, (rows, 288), 1) == tif_ref[:, :]).astype(_f32)
    d_oh = (lax.broadcasted_iota(jnp.int32, (rows, 8), 1) == dif_ref[:, :]).astype(_f32)
    t_emb = jnp.dot(t_oh, temb_ref[:, :], preferred_element_type=_f32)
    d_emb = jnp.dot(d_oh, demb_ref[:, :], preferred_element_type=_f32)
    h = jnp.zeros((rows, RNN_H), dtype=_f32)
    wih = wih_ref[:, :]
    bih = bih_ref[:, :]
    bhh = bhh_ref[:, :]
    for t in range(HIST_T):
        x_t = hist_ref[:, t:t + 1]
        gi = jnp.dot(x_t, wih, preferred_element_type=_f32) + bih
        gh = jnp.dot(h, whh_ref[:, :], preferred_element_type=_f32) + bhh
        i_r = gi[:, 0:RNN_H]
        i_z = gi[:, RNN_H:2 * RNN_H]
        i_n = gi[:, 2 * RNN_H:3 * RNN_H]
        h_r = gh[:, 0:RNN_H]
        h_z = gh[:, RNN_H:2 * RNN_H]
        h_n = gh[:, 2 * RNN_H:3 * RNN_H]
        r = 0.5 * (1.0 + jnp.tanh(0.5 * (i_r + h_r)))
        z = 0.5 * (1.0 + jnp.tanh(0.5 * (i_z + h_z)))
        nn_ = jnp.tanh(i_n + r * h_n)
        h = (1.0 - z) * nn_ + z * h
    feat = jnp.concatenate([xs_ref[:, :], t_emb, d_emb, h], axis=1)
    h1 = jnp.dot(feat, w1_ref[:, :], preferred_element_type=_f32)
    h1_ref[:, :] = h1
    asrc_ref[:, :] = jnp.dot(h1, asm_ref[:, :], preferred_element_type=_f32)
    adst_ref[:, :] = jnp.dot(h1, adm_ref[:, :], preferred_element_type=_f32)


def _run_k1(xs, hist, tif, dif, temb, demb, wih, whh, bih, bhh, w1, asm, adm):
    nblk = NPAD // R
    full = lambda shape: pl.BlockSpec(shape, lambda i: (0, 0))
    rowblk = lambda c: pl.BlockSpec((R, c), lambda i: (i, 0))
    return pl.pallas_call(
        _k1_body,
        grid=(nblk,),
        in_specs=[rowblk(16), rowblk(HIST_T), rowblk(1), rowblk(1),
                  full((288, 16)), full((8, 16)), full((1, 192)),
                  full((RNN_H, 192)), full((1, 192)), full((1, 192)),
                  full((GNN_IN, 128)), full((128, 16)), full((128, 16))],
        out_specs=[rowblk(128), rowblk(16), rowblk(16)],
        out_shape=[jax.ShapeDtypeStruct((NPAD, 128), _f32),
                   jax.ShapeDtypeStruct((NPAD, 16), _f32),
                   jax.ShapeDtypeStruct((NPAD, 16), _f32)],
    )(xs, hist, tif, dif, temb, demb, wih, whh, bih, bhh, w1, asm, adm)


def _mid_body(heads, scale, d0_ref, d1_ref, asrc_ref, adst_ref, h_ref,
              bdst_ref, selfm_ref):
    rows = d0_ref.shape[0]
    a_s = asrc_ref[:, 0:4]
    a_d = adst_ref[:, 0:4]
    t = a_s + a_d
    ex_self = jnp.exp(jnp.where(t > 0, t, 0.2 * t))
    denom = d0_ref[:, 0:4] + d1_ref[:, 0:4] + ex_self
    rdenom = scale / (denom + 1e-16)
    zs8 = jnp.zeros((rows, 8), dtype=_f32)
    bdst_ref[:, :] = jnp.concatenate([a_d, rdenom, zs8], axis=1)
    alpha_self = ex_self * rdenom
    acc = jnp.zeros((rows, GNN_H), dtype=_f32)
    for hd in range(heads):
        acc = acc + alpha_self[:, hd:hd + 1] * h_ref[:, hd * GNN_H:(hd + 1) * GNN_H]
    selfm_ref[:, :] = acc


def _run_mid(heads, scale, d0, d1, asrc, adst, h):
    nblk = NPAD // R
    hw = h.shape[1]
    rowblk = lambda c: pl.BlockSpec((R, c), lambda i: (i, 0))
    return pl.pallas_call(
        functools.partial(_mid_body, heads, scale),
        grid=(nblk,),
        in_specs=[rowblk(16), rowblk(16), rowblk(16), rowblk(16), rowblk(hw)],
        out_specs=[rowblk(16), rowblk(GNN_H)],
        out_shape=[jax.ShapeDtypeStruct((NPAD, 16), _f32),
                   jax.ShapeDtypeStruct((NPAD, GNN_H), _f32)],
    )(d0, d1, asrc, adst, h)


def _k3_body(p0_ref, p1_ref, selfm_ref, b1_ref, w2_ref, a2s_ref, a2d_ref,
             h2_ref, asrc_ref, adst_ref):
    rows = p0_ref.shape[0]
    x = p0_ref[:, :] + p1_ref[:, :] + selfm_ref[:, :] + b1_ref[:, :]
    x = jnp.maximum(x, 0.0)
    h2 = jnp.dot(x, w2_ref[:, :], preferred_element_type=_f32)
    h2_ref[:, :] = h2
    asrc_ref[:, :] = jnp.dot(h2, a2s_ref[:, :], preferred_element_type=_f32)
    adst_ref[:, :] = jnp.dot(h2, a2d_ref[:, :], preferred_element_type=_f32)


def _run_k3(p0, p1, selfm, b1, w2, a2s, a2d):
    nblk = NPAD // R
    full = lambda shape: pl.BlockSpec(shape, lambda i: (0, 0))
    rowblk = lambda c: pl.BlockSpec((R, c), lambda i: (i, 0))
    return pl.pallas_call(
        _k3_body,
        grid=(nblk,),
        in_specs=[rowblk(GNN_H), rowblk(GNN_H), rowblk(GNN_H),
                  full((1, GNN_H)), full((GNN_H, GNN_H)),
                  full((GNN_H, 16)), full((GNN_H, 16))],
        out_specs=[rowblk(GNN_H), rowblk(16), rowblk(16)],
        out_shape=[jax.ShapeDtypeStruct((NPAD, GNN_H), _f32),
                   jax.ShapeDtypeStruct((NPAD, 16), _f32),
                   jax.ShapeDtypeStruct((NPAD, 16), _f32)],
    )(p0, p1, selfm, b1, w2, a2s, a2d)


def _k5_body(p0_ref, p1_ref, selfm_ref, b2_ref, out_ref):
    x = p0_ref[:, :] + p1_ref[:, :] + selfm_ref[:, :] + b2_ref[:, :]
    out_ref[:, :] = jnp.maximum(x, 0.0)


def _run_k5(p0, p1, selfm, b2):
    nblk = NPAD // R
    full = lambda shape: pl.BlockSpec(shape, lambda i: (0, 0))
    rowblk = lambda c: pl.BlockSpec((R, c), lambda i: (i, 0))
    return pl.pallas_call(
        _k5_body,
        grid=(nblk,),
        in_specs=[rowblk(GNN_H), rowblk(GNN_H), rowblk(GNN_H), full((1, GNN_H))],
        out_specs=rowblk(GNN_H),
        out_shape=jax.ShapeDtypeStruct((NPAD, GNN_H), _f32),
    )(p0, p1, selfm, b2)


def _make_sc_pass(mode, hw=0, heads=0, w=W):
    nb = EP_TILE // w
    ow = 16 if mode == "a" else GNN_H
    mesh = plsc.VectorSubcoreMesh(core_axis_name="c", subcore_axis_name="s")
    scratch = (
        [pltpu.VMEM((w,), jnp.int32)] * 6 +
        [pltpu.VMEM((w, 16), _f32)] * 4 +
        ([pltpu.VMEM((w, hw), _f32)] * 2 if mode == "b" else []) +
        [pltpu.VMEM((w, ow), _f32)] * 2 +
        [pltpu.VMEM((64, ow), _f32)] +
        [pltpu.VMEM_SHARED((NPAD, ow), _f32)] +
        [pltpu.SemaphoreType.DMA] * 6
    )

    @functools.partial(
        pl.kernel, mesh=mesh,
        compiler_params=pltpu.CompilerParams(use_tc_tiling_on_sc=False),
        out_type=jax.ShapeDtypeStruct((2, NPAD, ow), _f32),
        scratch_types=scratch,
    )
    def kp(*refs):
        if mode == "a":
            (src_hbm, dst_hbm, tabs_hbm, tabd_hbm, out_hbm,
             sidx0, didx0, scidx0, sidx1, didx1, scidx1,
             srows0, drows0, srows1, drows1,
             orows0, orows1, zbuf, acc,
             gsem0, gsem1, idxsem0, idxsem1, scsem0, scsem1) = refs
            hrows = None
        else:
            (src_hbm, dst_hbm, tabs_hbm, tabd_hbm, h_hbm, out_hbm,
             sidx0, didx0, scidx0, sidx1, didx1, scidx1,
             srows0, drows0, srows1, drows1, hrows0, hrows1,
             orows0, orows1, zbuf, acc,
             gsem0, gsem1, idxsem0, idxsem1, scsem0, scsem1) = refs
            hrows = (hrows0, hrows1)
        sidx = (sidx0, sidx1)
        didx = (didx0, didx1)
        scidx = (scidx0, scidx1)
        srows = (srows0, srows1)
        drows = (drows0, drows1)
        orows = (orows0, orows1)
        gsem = (gsem0, gsem1)
        idxsem = (idxsem0, idxsem1)
        scsem = (scsem0, scsem1)

        cid = lax.axis_index("c")
        sid = lax.axis_index("s")
        wid = sid * 2 + cid
        base = wid * EP_TILE
        lane = lax.iota(jnp.int32, 16)
        exmask = lane < 4

        @pl.loop(0, 64)
        def _(i):
            for c in range(ow // 16):
                zbuf[i, pl.ds(c * 16, 16)] = jnp.zeros((16,), _f32)

        @pl.loop(0, PR // 64)
        def _(j):
            pltpu.sync_copy(zbuf, acc.at[pl.ds(sid * PR + j * 64, 64)])

        plsc.subcore_barrier()

        def issue_idx(wi, b):
            off = base + wi * w
            pltpu.async_copy(src_hbm.at[pl.ds(off, w)], sidx[b], idxsem[b])
            pltpu.async_copy(dst_hbm.at[pl.ds(off, w)], didx[b], idxsem[b])

        def drain_idx(b):
            pltpu.make_async_copy(src_hbm.at[pl.ds(0, w)], sidx[b],
                                  idxsem[b]).wait()
            pltpu.make_async_copy(dst_hbm.at[pl.ds(0, w)], didx[b],
                                  idxsem[b]).wait()

        def issue_gathers(b):
            pltpu.async_copy(tabs_hbm.at[sidx[b]], srows[b], gsem[b])
            pltpu.async_copy(tabd_hbm.at[didx[b]], drows[b], gsem[b])
            if mode == "b":
                pltpu.async_copy(h_hbm.at[sidx[b]], hrows[b], gsem[b])

        def drain_gathers(b):
            pltpu.make_async_copy(tabs_hbm.at[pl.ds(0, w)], srows[b],
                                  gsem[b]).wait()
            pltpu.make_async_copy(tabd_hbm.at[pl.ds(0, w)], drows[b],
                                  gsem[b]).wait()
            if mode == "b":
                pltpu.make_async_copy(h_hbm.at[pl.ds(0, w)], hrows[b],
                                      gsem[b]).wait()

        def drain_scatter(b):
            pltpu.make_async_copy(out_hbm.at[0, pl.ds(0, w)], orows[b],
                                  scsem[b]).wait()

        def compute(b):
            @plsc.parallel_loop(0, w, unroll=2)
            def _(e):
                t = srows[b].at[e][...] + drows[b].at[e][...]
                t = jnp.where(t > 0, t, 0.2 * t)
                if mode == "a":
                    orows[b].at[e][...] = jnp.exp(t)
                else:
                    v = jnp.where(exmask, jnp.exp(t), t)
                    hrow = hrows[b].at[e]
                    m0 = jnp.zeros((16,), _f32)
                    m1 = jnp.zeros((16,), _f32)
                    for hd in range(heads):
                        a_sc = v[hd] * v[4 + hd]
                        ab = lax.broadcast_in_dim(a_sc, (16,), ())
                        m0 = m0 + ab * hrow[pl.ds(hd * GNN_H, 16)]
                        m1 = m1 + ab * hrow[pl.ds(hd * GNN_H + 16, 16)]
                    orow = orows[b].at[e]
                    orow[pl.ds(0, 16)] = m0
                    orow[pl.ds(16, 16)] = m1

        pltpu.sync_copy(src_hbm.at[pl.ds(base, w)], sidx[0])
        pltpu.sync_copy(dst_hbm.at[pl.ds(base, w)], didx[0])
        issue_gathers(0)
        issue_idx(1, 1)

        @pl.loop(0, nb // 2)
        def _(tt):
            for b in range(2):
                o = 1 - b
                wi = tt * 2 + b

                @pl.when(wi + 1 < nb)
                def _():
                    drain_idx(o)
                    issue_gathers(o)

                drain_gathers(b)

                @pl.when(wi >= 2)
                def _():
                    drain_scatter(b)

                for k in range(w // 16):
                    scidx[b][pl.ds(k * 16, 16)] = didx[b][pl.ds(k * 16, 16)]

                @pl.when(wi + 2 < nb)
                def _():
                    issue_idx(wi + 2, b)

                compute(b)
                pltpu.async_copy(orows[b], acc.at[scidx[b]], scsem[b],
                                 add=True)

        drain_scatter(0)
        drain_scatter(1)
        plsc.subcore_barrier()
        pltpu.sync_copy(acc.at[pl.ds(sid * PR, PR)],
                        out_hbm.at[cid, pl.ds(sid * PR, PR)])

    return kp


_sc_pass_a = _make_sc_pass("a")
_sc_pass_b1 = _make_sc_pass("b", 128, HEADS1, w=64)
_sc_pass_b2 = _make_sc_pass("b", GNN_H, 1)


def kernel(x_static, hist_speed, time_idx, day_idx, edge_index, time_emb,
           day_emb, gru_W_ih, gru_W_hh, gru_b_ih, gru_b_hh, gat1_W,
           gat1_att_src, gat1_att_dst, gat1_bias, gat2_W, gat2_att_src,
           gat2_att_dst, gat2_bias):
    pad_n = NPAD - N
    xs = jnp.pad(x_static, ((0, pad_n), (0, 0)))
    hist = jnp.pad(hist_speed.reshape(N, HIST_T), ((0, pad_n), (0, 0)))
    tif = jnp.pad(time_idx.astype(jnp.int32).reshape(N, 1), ((0, pad_n), (0, 0)))
    dif = jnp.pad(day_idx.astype(jnp.int32).reshape(N, 1), ((0, pad_n), (0, 0)))
    demb = jnp.pad(day_emb, ((0, 1), (0, 0)))
    wih = gru_W_ih.reshape(1, 3 * RNN_H)
    whh = gru_W_hh.T
    bih = gru_b_ih.reshape(1, 3 * RNN_H)
    bhh = gru_b_hh.reshape(1, 3 * RNN_H)
    headmask = (jnp.arange(HEADS1 * GNN_H)[:, None] // GNN_H
                == jnp.arange(16)[None, :]).astype(_f32)
    a1s = headmask * gat1_att_src.reshape(HEADS1 * GNN_H, 1)
    a1d = headmask * gat1_att_dst.reshape(HEADS1 * GNN_H, 1)
    col0 = (jnp.arange(16)[None, :] == 0).astype(_f32)
    a2s = gat2_att_src.reshape(GNN_H, 1) * col0
    a2d = gat2_att_dst.reshape(GNN_H, 1) * col0
    b1 = gat1_bias.reshape(1, GNN_H)
    b2 = gat2_bias.reshape(1, GNN_H)

    pad_e = E_PAD - E
    fill = jnp.full((pad_e,), NPAD - 1, jnp.int32)
    src = jnp.concatenate([edge_index[0].astype(jnp.int32), fill])
    dst = jnp.concatenate([edge_index[1].astype(jnp.int32), fill])

    h1, asrc1, adst1 = _run_k1(xs, hist, tif, dif, time_emb, demb, wih, whh,
                               bih, bhh, gat1_W, a1s, a1d)
    dpart1 = _sc_pass_a(src, dst, asrc1, adst1)
    bdst1, selfm1 = _run_mid(HEADS1, 0.25, dpart1[0], dpart1[1],
                             asrc1, adst1, h1)
    opart1 = _sc_pass_b1(src, dst, asrc1, bdst1, h1)
    h2, asrc2, adst2 = _run_k3(opart1[0], opart1[1], selfm1, b1,
                               gat2_W, a2s, a2d)
    dpart2 = _sc_pass_a(src, dst, asrc2, adst2)
    bdst2, selfm2 = _run_mid(1, 1.0, dpart2[0], dpart2[1], asrc2, adst2, h2)
    opart2 = _sc_pass_b2(src, dst, asrc2, bdst2, h2)
    out = _run_k5(opart2[0], opart2[1], selfm2, b2)
    return out[:N]


# speedup vs baseline: 62.8139x; 1.0005x over previous
import functools

import jax
import jax.numpy as jnp
from jax import lax
from jax.experimental import pallas as pl
from jax.experimental.pallas import tpu as pltpu
from jax.experimental.pallas import tpu_sc as plsc

N = 50000
E = 800000
NPAD = 50176
R = 1024
HIST_T = 12
RNN_H = 64
GNN_H = 32
HEADS1 = 4
GNN_IN = 112

NW = 32
W = 128
E_PAD = 802816
EP_TILE = E_PAD // NW
NB = EP_TILE // W
PR = NPAD // 16

_f32 = jnp.float32


def _k1_body(xs_ref, hist_ref, tif_ref, dif_ref,
             temb_ref, demb_ref, wih_ref, whh_ref, bih_ref, bhh_ref,
             w1_ref, asm_ref, adm_ref,
             h1_ref, asrc_ref, adst_ref):
    rows = xs_ref.shape[0]
    t_oh = (lax.broadcasted_iota(jnp.int32, (rows, 288), 1) == tif_ref[:, :]).astype(_f32)
    d_oh = (lax.broadcasted_iota(jnp.int32, (rows, 8), 1) == dif_ref[:, :]).astype(_f32)
    t_emb = jnp.dot(t_oh, temb_ref[:, :], preferred_element_type=_f32)
    d_emb = jnp.dot(d_oh, demb_ref[:, :], preferred_element_type=_f32)
    h = jnp.zeros((rows, RNN_H), dtype=_f32)
    wih = wih_ref[:, :]
    bih = bih_ref[:, :]
    bhh = bhh_ref[:, :]
    for t in range(HIST_T):
        x_t = hist_ref[:, t:t + 1]
        gi = jnp.dot(x_t, wih, preferred_element_type=_f32) + bih
        gh = jnp.dot(h, whh_ref[:, :], preferred_element_type=_f32) + bhh
        i_r = gi[:, 0:RNN_H]
        i_z = gi[:, RNN_H:2 * RNN_H]
        i_n = gi[:, 2 * RNN_H:3 * RNN_H]
        h_r = gh[:, 0:RNN_H]
        h_z = gh[:, RNN_H:2 * RNN_H]
        h_n = gh[:, 2 * RNN_H:3 * RNN_H]
        r = 0.5 * (1.0 + jnp.tanh(0.5 * (i_r + h_r)))
        z = 0.5 * (1.0 + jnp.tanh(0.5 * (i_z + h_z)))
        nn_ = jnp.tanh(i_n + r * h_n)
        h = (1.0 - z) * nn_ + z * h
    feat = jnp.concatenate([xs_ref[:, :], t_emb, d_emb, h], axis=1)
    h1 = jnp.dot(feat, w1_ref[:, :], preferred_element_type=_f32)
    h1_ref[:, :] = h1
    asrc_ref[:, :] = jnp.dot(h1, asm_ref[:, :], preferred_element_type=_f32)
    adst_ref[:, :] = jnp.dot(h1, adm_ref[:, :], preferred_element_type=_f32)


def _run_k1(xs, hist, tif, dif, temb, demb, wih, whh, bih, bhh, w1, asm, adm):
    nblk = NPAD // R
    full = lambda shape: pl.BlockSpec(shape, lambda i: (0, 0))
    rowblk = lambda c: pl.BlockSpec((R, c), lambda i: (i, 0))
    return pl.pallas_call(
        _k1_body,
        grid=(nblk,),
        in_specs=[rowblk(16), rowblk(HIST_T), rowblk(1), rowblk(1),
                  full((288, 16)), full((8, 16)), full((1, 192)),
                  full((RNN_H, 192)), full((1, 192)), full((1, 192)),
                  full((GNN_IN, 128)), full((128, 16)), full((128, 16))],
        out_specs=[rowblk(128), rowblk(16), rowblk(16)],
        out_shape=[jax.ShapeDtypeStruct((NPAD, 128), _f32),
                   jax.ShapeDtypeStruct((NPAD, 16), _f32),
                   jax.ShapeDtypeStruct((NPAD, 16), _f32)],
    )(xs, hist, tif, dif, temb, demb, wih, whh, bih, bhh, w1, asm, adm)


def _mid_body(heads, scale, d0_ref, d1_ref, asrc_ref, adst_ref, h_ref,
              bdst_ref, selfm_ref):
    rows = d0_ref.shape[0]
    a_s = asrc_ref[:, 0:4]
    a_d = adst_ref[:, 0:4]
    t = a_s + a_d
    ex_self = jnp.exp(jnp.where(t > 0, t, 0.2 * t))
    denom = d0_ref[:, 0:4] + d1_ref[:, 0:4] + ex_self
    rdenom = scale / (denom + 1e-16)
    zs8 = jnp.zeros((rows, 8), dtype=_f32)
    bdst_ref[:, :] = jnp.concatenate([a_d, rdenom, zs8], axis=1)
    alpha_self = ex_self * rdenom
    acc = jnp.zeros((rows, GNN_H), dtype=_f32)
    for hd in range(heads):
        acc = acc + alpha_self[:, hd:hd + 1] * h_ref[:, hd * GNN_H:(hd + 1) * GNN_H]
    selfm_ref[:, :] = acc


def _run_mid(heads, scale, d0, d1, asrc, adst, h):
    nblk = NPAD // R
    hw = h.shape[1]
    rowblk = lambda c: pl.BlockSpec((R, c), lambda i: (i, 0))
    return pl.pallas_call(
        functools.partial(_mid_body, heads, scale),
        grid=(nblk,),
        in_specs=[rowblk(16), rowblk(16), rowblk(16), rowblk(16), rowblk(hw)],
        out_specs=[rowblk(16), rowblk(GNN_H)],
        out_shape=[jax.ShapeDtypeStruct((NPAD, 16), _f32),
                   jax.ShapeDtypeStruct((NPAD, GNN_H), _f32)],
    )(d0, d1, asrc, adst, h)


def _k3_body(p0_ref, p1_ref, selfm_ref, b1_ref, w2_ref, a2s_ref, a2d_ref,
             h2_ref, asrc_ref, adst_ref):
    rows = p0_ref.shape[0]
    x = p0_ref[:, :] + p1_ref[:, :] + selfm_ref[:, :] + b1_ref[:, :]
    x = jnp.maximum(x, 0.0)
    h2 = jnp.dot(x, w2_ref[:, :], preferred_element_type=_f32)
    h2_ref[:, :] = h2
    asrc_ref[:, :] = jnp.dot(h2, a2s_ref[:, :], preferred_element_type=_f32)
    adst_ref[:, :] = jnp.dot(h2, a2d_ref[:, :], preferred_element_type=_f32)


def _run_k3(p0, p1, selfm, b1, w2, a2s, a2d):
    nblk = NPAD // R
    full = lambda shape: pl.BlockSpec(shape, lambda i: (0, 0))
    rowblk = lambda c: pl.BlockSpec((R, c), lambda i: (i, 0))
    return pl.pallas_call(
        _k3_body,
        grid=(nblk,),
        in_specs=[rowblk(GNN_H), rowblk(GNN_H), rowblk(GNN_H),
                  full((1, GNN_H)), full((GNN_H, GNN_H)),
                  full((GNN_H, 16)), full((GNN_H, 16))],
        out_specs=[rowblk(GNN_H), rowblk(16), rowblk(16)],
        out_shape=[jax.ShapeDtypeStruct((NPAD, GNN_H), _f32),
                   jax.ShapeDtypeStruct((NPAD, 16), _f32),
                   jax.ShapeDtypeStruct((NPAD, 16), _f32)],
    )(p0, p1, selfm, b1, w2, a2s, a2d)


def _k5_body(p0_ref, p1_ref, selfm_ref, b2_ref, out_ref):
    x = p0_ref[:, :] + p1_ref[:, :] + selfm_ref[:, :] + b2_ref[:, :]
    out_ref[:, :] = jnp.maximum(x, 0.0)


def _run_k5(p0, p1, selfm, b2):
    nblk = NPAD // R
    full = lambda shape: pl.BlockSpec(shape, lambda i: (0, 0))
    rowblk = lambda c: pl.BlockSpec((R, c), lambda i: (i, 0))
    return pl.pallas_call(
        _k5_body,
        grid=(nblk,),
        in_specs=[rowblk(GNN_H), rowblk(GNN_H), rowblk(GNN_H), full((1, GNN_H))],
        out_specs=rowblk(GNN_H),
        out_shape=jax.ShapeDtypeStruct((NPAD, GNN_H), _f32),
    )(p0, p1, selfm, b2)


def _make_sc_pass(mode, hw=0, heads=0, w=W):
    nb = EP_TILE // w
    ow = 16 if mode == "a" else GNN_H
    mesh = plsc.VectorSubcoreMesh(core_axis_name="c", subcore_axis_name="s")
    scratch = (
        [pltpu.VMEM((w,), jnp.int32)] * 6 +
        [pltpu.VMEM((w, 16), _f32)] * 4 +
        ([pltpu.VMEM((w, hw), _f32)] * 2 if mode == "b" else []) +
        [pltpu.VMEM((w, ow), _f32)] * 2 +
        [pltpu.VMEM((64, ow), _f32)] +
        [pltpu.VMEM_SHARED((NPAD, ow), _f32)] +
        [pltpu.SemaphoreType.DMA] * 6
    )

    @functools.partial(
        pl.kernel, mesh=mesh,
        compiler_params=pltpu.CompilerParams(use_tc_tiling_on_sc=False),
        out_type=jax.ShapeDtypeStruct((2, NPAD, ow), _f32),
        scratch_types=scratch,
    )
    def kp(*refs):
        if mode == "a":
            (src_hbm, dst_hbm, tabs_hbm, tabd_hbm, out_hbm,
             sidx0, didx0, scidx0, sidx1, didx1, scidx1,
             srows0, drows0, srows1, drows1,
             orows0, orows1, zbuf, acc,
             gsem0, gsem1, idxsem0, idxsem1, scsem0, scsem1) = refs
            hrows = None
        else:
            (src_hbm, dst_hbm, tabs_hbm, tabd_hbm, h_hbm, out_hbm,
             sidx0, didx0, scidx0, sidx1, didx1, scidx1,
             srows0, drows0, srows1, drows1, hrows0, hrows1,
             orows0, orows1, zbuf, acc,
             gsem0, gsem1, idxsem0, idxsem1, scsem0, scsem1) = refs
            hrows = (hrows0, hrows1)
        sidx = (sidx0, sidx1)
        didx = (didx0, didx1)
        scidx = (scidx0, scidx1)
        srows = (srows0, srows1)
        drows = (drows0, drows1)
        orows = (orows0, orows1)
        gsem = (gsem0, gsem1)
        idxsem = (idxsem0, idxsem1)
        scsem = (scsem0, scsem1)

        cid = lax.axis_index("c")
        sid = lax.axis_index("s")
        wid = sid * 2 + cid
        base = wid * EP_TILE
        lane = lax.iota(jnp.int32, 16)
        exmask = lane < 4

        @pl.loop(0, 64)
        def _(i):
            for c in range(ow // 16):
                zbuf[i, pl.ds(c * 16, 16)] = jnp.zeros((16,), _f32)

        @pl.loop(0, PR // 64)
        def _(j):
            pltpu.sync_copy(zbuf, acc.at[pl.ds(sid * PR + j * 64, 64)])

        plsc.subcore_barrier()

        def issue_idx(wi, b):
            off = base + wi * w
            pltpu.async_copy(src_hbm.at[pl.ds(off, w)], sidx[b], idxsem[b])
            pltpu.async_copy(dst_hbm.at[pl.ds(off, w)], didx[b], idxsem[b])

        def drain_idx(b):
            pltpu.make_async_copy(src_hbm.at[pl.ds(0, w)], sidx[b],
                                  idxsem[b]).wait()
            pltpu.make_async_copy(dst_hbm.at[pl.ds(0, w)], didx[b],
                                  idxsem[b]).wait()

        def issue_gathers(b):
            pltpu.async_copy(tabs_hbm.at[sidx[b]], srows[b], gsem[b])
            pltpu.async_copy(tabd_hbm.at[didx[b]], drows[b], gsem[b])
            if mode == "b":
                pltpu.async_copy(h_hbm.at[sidx[b]], hrows[b], gsem[b])

        def drain_gathers(b):
            pltpu.make_async_copy(tabs_hbm.at[pl.ds(0, w)], srows[b],
                                  gsem[b]).wait()
            pltpu.make_async_copy(tabd_hbm.at[pl.ds(0, w)], drows[b],
                                  gsem[b]).wait()
            if mode == "b":
                pltpu.make_async_copy(h_hbm.at[pl.ds(0, w)], hrows[b],
                                      gsem[b]).wait()

        def drain_scatter(b):
            pltpu.make_async_copy(out_hbm.at[0, pl.ds(0, w)], orows[b],
                                  scsem[b]).wait()

        def compute(b):
            @plsc.parallel_loop(0, w, unroll=4)
            def _(e):
                t = srows[b].at[e][...] + drows[b].at[e][...]
                t = jnp.where(t > 0, t, 0.2 * t)
                if mode == "a":
                    orows[b].at[e][...] = jnp.exp(t)
                else:
                    v = jnp.where(exmask, jnp.exp(t), t)
                    hrow = hrows[b].at[e]
                    m0 = jnp.zeros((16,), _f32)
                    m1 = jnp.zeros((16,), _f32)
                    for hd in range(heads):
                        a_sc = v[hd] * v[4 + hd]
                        ab = lax.broadcast_in_dim(a_sc, (16,), ())
                        m0 = m0 + ab * hrow[pl.ds(hd * GNN_H, 16)]
                        m1 = m1 + ab * hrow[pl.ds(hd * GNN_H + 16, 16)]
                    orow = orows[b].at[e]
                    orow[pl.ds(0, 16)] = m0
                    orow[pl.ds(16, 16)] = m1

        pltpu.sync_copy(src_hbm.at[pl.ds(base, w)], sidx[0])
        pltpu.sync_copy(dst_hbm.at[pl.ds(base, w)], didx[0])
        issue_gathers(0)
        issue_idx(1, 1)

        @pl.loop(0, nb // 2)
        def _(tt):
            for b in range(2):
                o = 1 - b
                wi = tt * 2 + b

                @pl.when(wi + 1 < nb)
                def _():
                    drain_idx(o)
                    issue_gathers(o)

                drain_gathers(b)

                @pl.when(wi >= 2)
                def _():
                    drain_scatter(b)

                for k in range(w // 16):
                    scidx[b][pl.ds(k * 16, 16)] = didx[b][pl.ds(k * 16, 16)]

                @pl.when(wi + 2 < nb)
                def _():
                    issue_idx(wi + 2, b)

                compute(b)
                pltpu.async_copy(orows[b], acc.at[scidx[b]], scsem[b],
                                 add=True)

        drain_scatter(0)
        drain_scatter(1)
        plsc.subcore_barrier()
        pltpu.sync_copy(acc.at[pl.ds(sid * PR, PR)],
                        out_hbm.at[cid, pl.ds(sid * PR, PR)])

    return kp


_sc_pass_a = _make_sc_pass("a")
_sc_pass_b1 = _make_sc_pass("b", 128, HEADS1, w=64)
_sc_pass_b2 = _make_sc_pass("b", GNN_H, 1)


def kernel(x_static, hist_speed, time_idx, day_idx, edge_index, time_emb,
           day_emb, gru_W_ih, gru_W_hh, gru_b_ih, gru_b_hh, gat1_W,
           gat1_att_src, gat1_att_dst, gat1_bias, gat2_W, gat2_att_src,
           gat2_att_dst, gat2_bias):
    pad_n = NPAD - N
    xs = jnp.pad(x_static, ((0, pad_n), (0, 0)))
    hist = jnp.pad(hist_speed.reshape(N, HIST_T), ((0, pad_n), (0, 0)))
    tif = jnp.pad(time_idx.astype(jnp.int32).reshape(N, 1), ((0, pad_n), (0, 0)))
    dif = jnp.pad(day_idx.astype(jnp.int32).reshape(N, 1), ((0, pad_n), (0, 0)))
    demb = jnp.pad(day_emb, ((0, 1), (0, 0)))
    wih = gru_W_ih.reshape(1, 3 * RNN_H)
    whh = gru_W_hh.T
    bih = gru_b_ih.reshape(1, 3 * RNN_H)
    bhh = gru_b_hh.reshape(1, 3 * RNN_H)
    headmask = (jnp.arange(HEADS1 * GNN_H)[:, None] // GNN_H
                == jnp.arange(16)[None, :]).astype(_f32)
    a1s = headmask * gat1_att_src.reshape(HEADS1 * GNN_H, 1)
    a1d = headmask * gat1_att_dst.reshape(HEADS1 * GNN_H, 1)
    col0 = (jnp.arange(16)[None, :] == 0).astype(_f32)
    a2s = gat2_att_src.reshape(GNN_H, 1) * col0
    a2d = gat2_att_dst.reshape(GNN_H, 1) * col0
    b1 = gat1_bias.reshape(1, GNN_H)
    b2 = gat2_bias.reshape(1, GNN_H)

    pad_e = E_PAD - E
    fill = jnp.full((pad_e,), NPAD - 1, jnp.int32)
    src = jnp.concatenate([edge_index[0].astype(jnp.int32), fill])
    dst = jnp.concatenate([edge_index[1].astype(jnp.int32), fill])

    h1, asrc1, adst1 = _run_k1(xs, hist, tif, dif, time_emb, demb, wih, whh,
                               bih, bhh, gat1_W, a1s, a1d)
    dpart1 = _sc_pass_a(src, dst, asrc1, adst1)
    bdst1, selfm1 = _run_mid(HEADS1, 0.25, dpart1[0], dpart1[1],
                             asrc1, adst1, h1)
    opart1 = _sc_pass_b1(src, dst, asrc1, bdst1, h1)
    h2, asrc2, adst2 = _run_k3(opart1[0], opart1[1], selfm1, b1,
                               gat2_W, a2s, a2d)
    dpart2 = _sc_pass_a(src, dst, asrc2, adst2)
    bdst2, selfm2 = _run_mid(1, 1.0, dpart2[0], dpart2[1], asrc2, adst2, h2)
    opart2 = _sc_pass_b2(src, dst, asrc2, bdst2, h2)
    out = _run_k5(opart2[0], opart2[1], selfm2, b2)
    return out[:N]


# speedup vs baseline: 64.8492x; 1.0324x over previous
import functools

import jax
import jax.numpy as jnp
from jax import lax
from jax.experimental import pallas as pl
from jax.experimental.pallas import tpu as pltpu
from jax.experimental.pallas import tpu_sc as plsc

N = 50000
E = 800000
NPAD = 50176
R = 1024
HIST_T = 12
RNN_H = 64
GNN_H = 32
HEADS1 = 4
GNN_IN = 112

NW = 32
W = 128
E_PAD = 802816
EP_TILE = E_PAD // NW
NB = EP_TILE // W
PR = NPAD // 16

_f32 = jnp.float32


def _k1_body(xs_ref, hist_ref, tif_ref, dif_ref,
             temb_ref, demb_ref, wih_ref, whh_ref, bih_ref, bhh_ref,
             w1_ref, asm_ref, adm_ref,
             h1_ref, asrc_ref, adst_ref):
    rows = xs_ref.shape[0]
    t_oh = (lax.broadcasted_iota(jnp.int32, (rows, 288), 1) == tif_ref[:, :]).astype(_f32)
    d_oh = (lax.broadcasted_iota(jnp.int32, (rows, 8), 1) == dif_ref[:, :]).astype(_f32)
    t_emb = jnp.dot(t_oh, temb_ref[:, :], preferred_element_type=_f32)
    d_emb = jnp.dot(d_oh, demb_ref[:, :], preferred_element_type=_f32)
    h = jnp.zeros((rows, RNN_H), dtype=_f32)
    wih = wih_ref[:, :]
    bih = bih_ref[:, :]
    bhh = bhh_ref[:, :]
    for t in range(HIST_T):
        x_t = hist_ref[:, t:t + 1]
        gi = jnp.dot(x_t, wih, preferred_element_type=_f32) + bih
        gh = jnp.dot(h, whh_ref[:, :], preferred_element_type=_f32) + bhh
        i_r = gi[:, 0:RNN_H]
        i_z = gi[:, RNN_H:2 * RNN_H]
        i_n = gi[:, 2 * RNN_H:3 * RNN_H]
        h_r = gh[:, 0:RNN_H]
        h_z = gh[:, RNN_H:2 * RNN_H]
        h_n = gh[:, 2 * RNN_H:3 * RNN_H]
        r = 0.5 * (1.0 + jnp.tanh(0.5 * (i_r + h_r)))
        z = 0.5 * (1.0 + jnp.tanh(0.5 * (i_z + h_z)))
        nn_ = jnp.tanh(i_n + r * h_n)
        h = (1.0 - z) * nn_ + z * h
    feat = jnp.concatenate([xs_ref[:, :], t_emb, d_emb, h], axis=1)
    h1 = jnp.dot(feat, w1_ref[:, :], preferred_element_type=_f32)
    h1_ref[:, :] = h1
    asrc_ref[:, :] = jnp.dot(h1, asm_ref[:, :], preferred_element_type=_f32)
    adst_ref[:, :] = jnp.dot(h1, adm_ref[:, :], preferred_element_type=_f32)


def _run_k1(xs, hist, tif, dif, temb, demb, wih, whh, bih, bhh, w1, asm, adm):
    nblk = NPAD // R
    full = lambda shape: pl.BlockSpec(shape, lambda i: (0, 0))
    rowblk = lambda c: pl.BlockSpec((R, c), lambda i: (i, 0))
    return pl.pallas_call(
        _k1_body,
        grid=(nblk,),
        in_specs=[rowblk(16), rowblk(HIST_T), rowblk(1), rowblk(1),
                  full((288, 16)), full((8, 16)), full((1, 192)),
                  full((RNN_H, 192)), full((1, 192)), full((1, 192)),
                  full((GNN_IN, 128)), full((128, 16)), full((128, 16))],
        out_specs=[rowblk(128), rowblk(16), rowblk(16)],
        out_shape=[jax.ShapeDtypeStruct((NPAD, 128), _f32),
                   jax.ShapeDtypeStruct((NPAD, 16), _f32),
                   jax.ShapeDtypeStruct((NPAD, 16), _f32)],
    )(xs, hist, tif, dif, temb, demb, wih, whh, bih, bhh, w1, asm, adm)


def _mid_body(heads, scale, sigma, d0_ref, d1_ref, asrc_ref, adst_ref, h_ref,
              bdst_ref, selfm_ref):
    rows = d0_ref.shape[0]
    a_s = asrc_ref[:, 0:4]
    a_d = adst_ref[:, 0:4]
    t = a_s + a_d
    ex_self = jnp.exp(jnp.where(t > 0, t, 0.2 * t))
    denom = d0_ref[:, 0:4] + d1_ref[:, 0:4] + ex_self
    rdenom = scale / (denom + 1e-16)
    zs8 = jnp.zeros((rows, 8), dtype=_f32)
    bdst_ref[:, :] = jnp.concatenate([a_d, rdenom, zs8], axis=1)
    alpha_self = ex_self * rdenom
    if sigma:
        ev = jnp.zeros((rows, GNN_H // 2), dtype=_f32)
        od = jnp.zeros((rows, GNN_H // 2), dtype=_f32)
        for hd in range(heads):
            a = alpha_self[:, hd:hd + 1]
            ev = ev + a * h_ref[:, hd * GNN_H:hd * GNN_H + GNN_H:2]
            od = od + a * h_ref[:, hd * GNN_H + 1:hd * GNN_H + GNN_H:2]
        selfm_ref[:, :] = jnp.concatenate([ev, od], axis=1)
    else:
        acc = jnp.zeros((rows, GNN_H), dtype=_f32)
        for hd in range(heads):
            acc = acc + alpha_self[:, hd:hd + 1] * h_ref[:, hd * GNN_H:(hd + 1) * GNN_H]
        selfm_ref[:, :] = acc


def _run_mid(heads, scale, sigma, d0, d1, asrc, adst, h):
    nblk = NPAD // R
    hw = h.shape[1]
    rowblk = lambda c: pl.BlockSpec((R, c), lambda i: (i, 0))
    return pl.pallas_call(
        functools.partial(_mid_body, heads, scale, sigma),
        grid=(nblk,),
        in_specs=[rowblk(16), rowblk(16), rowblk(16), rowblk(16), rowblk(hw)],
        out_specs=[rowblk(16), rowblk(GNN_H)],
        out_shape=[jax.ShapeDtypeStruct((NPAD, 16), _f32),
                   jax.ShapeDtypeStruct((NPAD, GNN_H), _f32)],
    )(d0, d1, asrc, adst, h)


def _k3_body(p0_ref, p1_ref, selfm_ref, b1_ref, w2_ref, a2s_ref, a2d_ref,
             h2_ref, asrc_ref, adst_ref):
    rows = p0_ref.shape[0]
    x = p0_ref[:, :] + p1_ref[:, :] + selfm_ref[:, :] + b1_ref[:, :]
    x = jnp.maximum(x, 0.0)
    h2 = jnp.dot(x, w2_ref[:, :], preferred_element_type=_f32)
    h2_ref[:, :] = h2
    asrc_ref[:, :] = jnp.dot(h2, a2s_ref[:, :], preferred_element_type=_f32)
    adst_ref[:, :] = jnp.dot(h2, a2d_ref[:, :], preferred_element_type=_f32)


def _run_k3(p0, p1, selfm, b1, w2, a2s, a2d):
    nblk = NPAD // R
    full = lambda shape: pl.BlockSpec(shape, lambda i: (0, 0))
    rowblk = lambda c: pl.BlockSpec((R, c), lambda i: (i, 0))
    return pl.pallas_call(
        _k3_body,
        grid=(nblk,),
        in_specs=[rowblk(GNN_H), rowblk(GNN_H), rowblk(GNN_H),
                  full((1, GNN_H)), full((GNN_H, GNN_H)),
                  full((GNN_H, 16)), full((GNN_H, 16))],
        out_specs=[rowblk(GNN_H), rowblk(16), rowblk(16)],
        out_shape=[jax.ShapeDtypeStruct((NPAD, GNN_H), _f32),
                   jax.ShapeDtypeStruct((NPAD, 16), _f32),
                   jax.ShapeDtypeStruct((NPAD, 16), _f32)],
    )(p0, p1, selfm, b1, w2, a2s, a2d)


def _k5_body(p0_ref, p1_ref, selfm_ref, b2_ref, out_ref):
    x = p0_ref[:, :] + p1_ref[:, :] + selfm_ref[:, :] + b2_ref[:, :]
    out_ref[:, :] = jnp.maximum(x, 0.0)


def _run_k5(p0, p1, selfm, b2):
    nblk = NPAD // R
    full = lambda shape: pl.BlockSpec(shape, lambda i: (0, 0))
    rowblk = lambda c: pl.BlockSpec((R, c), lambda i: (i, 0))
    return pl.pallas_call(
        _k5_body,
        grid=(nblk,),
        in_specs=[rowblk(GNN_H), rowblk(GNN_H), rowblk(GNN_H), full((1, GNN_H))],
        out_specs=rowblk(GNN_H),
        out_shape=jax.ShapeDtypeStruct((NPAD, GNN_H), _f32),
    )(p0, p1, selfm, b2)


def _make_sc_pass(mode, hw=0, heads=0, w=W):
    nb = EP_TILE // w
    ow = 16 if mode == "a" else GNN_H
    mesh = plsc.VectorSubcoreMesh(core_axis_name="c", subcore_axis_name="s")
    scratch = (
        [pltpu.VMEM((w,), jnp.int32)] * 6 +
        [pltpu.VMEM((w, 16), _f32)] * 4 +
        ([pltpu.VMEM((w, hw), _f32)] * 2 if mode == "b" else []) +
        [pltpu.VMEM((w, ow), _f32)] * 2 +
        [pltpu.VMEM((64, ow), _f32)] +
        [pltpu.VMEM_SHARED((NPAD, ow), _f32)] +
        [pltpu.SemaphoreType.DMA] * 6
    )

    @functools.partial(
        pl.kernel, mesh=mesh,
        compiler_params=pltpu.CompilerParams(use_tc_tiling_on_sc=False),
        out_type=jax.ShapeDtypeStruct((2, NPAD, ow), _f32),
        scratch_types=scratch,
    )
    def kp(*refs):
        if mode == "a":
            (src_hbm, dst_hbm, tabs_hbm, tabd_hbm, out_hbm,
             sidx0, didx0, scidx0, sidx1, didx1, scidx1,
             srows0, drows0, srows1, drows1,
             orows0, orows1, zbuf, acc,
             gsem0, gsem1, idxsem0, idxsem1, scsem0, scsem1) = refs
            hrows = None
        else:
            (src_hbm, dst_hbm, tabs_hbm, tabd_hbm, h_hbm, out_hbm,
             sidx0, didx0, scidx0, sidx1, didx1, scidx1,
             srows0, drows0, srows1, drows1, hrows0, hrows1,
             orows0, orows1, zbuf, acc,
             gsem0, gsem1, idxsem0, idxsem1, scsem0, scsem1) = refs
            hrows = (hrows0, hrows1)
        sidx = (sidx0, sidx1)
        didx = (didx0, didx1)
        scidx = (scidx0, scidx1)
        srows = (srows0, srows1)
        drows = (drows0, drows1)
        orows = (orows0, orows1)
        gsem = (gsem0, gsem1)
        idxsem = (idxsem0, idxsem1)
        scsem = (scsem0, scsem1)

        cid = lax.axis_index("c")
        sid = lax.axis_index("s")
        wid = sid * 2 + cid
        base = wid * EP_TILE
        lane = lax.iota(jnp.int32, 16)
        exmask = lane < 4

        @pl.loop(0, 64)
        def _(i):
            for c in range(ow // 16):
                zbuf[i, pl.ds(c * 16, 16)] = jnp.zeros((16,), _f32)

        @pl.loop(0, PR // 64)
        def _(j):
            pltpu.sync_copy(zbuf, acc.at[pl.ds(sid * PR + j * 64, 64)])

        plsc.subcore_barrier()

        def issue_idx(wi, b):
            off = base + wi * w
            pltpu.async_copy(src_hbm.at[pl.ds(off, w)], sidx[b], idxsem[b])
            pltpu.async_copy(dst_hbm.at[pl.ds(off, w)], didx[b], idxsem[b])

        def drain_idx(b):
            pltpu.make_async_copy(src_hbm.at[pl.ds(0, w)], sidx[b],
                                  idxsem[b]).wait()
            pltpu.make_async_copy(dst_hbm.at[pl.ds(0, w)], didx[b],
                                  idxsem[b]).wait()

        def issue_gathers(b):
            pltpu.async_copy(tabs_hbm.at[sidx[b]], srows[b], gsem[b])
            pltpu.async_copy(tabd_hbm.at[didx[b]], drows[b], gsem[b])
            if mode == "b":
                pltpu.async_copy(h_hbm.at[sidx[b]], hrows[b], gsem[b])

        def drain_gathers(b):
            pltpu.make_async_copy(tabs_hbm.at[pl.ds(0, w)], srows[b],
                                  gsem[b]).wait()
            pltpu.make_async_copy(tabd_hbm.at[pl.ds(0, w)], drows[b],
                                  gsem[b]).wait()
            if mode == "b":
                pltpu.make_async_copy(h_hbm.at[pl.ds(0, w)], hrows[b],
                                      gsem[b]).wait()

        def drain_scatter(b):
            pltpu.make_async_copy(out_hbm.at[0, pl.ds(0, w)], orows[b],
                                  scsem[b]).wait()

        def compute(b):
            @plsc.parallel_loop(0, w, unroll=4)
            def _(e):
                t = srows[b].at[e][...] + drows[b].at[e][...]
                t = jnp.where(t > 0, t, 0.2 * t)
                if mode == "a":
                    orows[b].at[e][...] = jnp.exp(t)
                else:
                    v = jnp.where(exmask, jnp.exp(t), t)
                    hrow = hrows[b].at[e]
                    m0 = jnp.zeros((16,), _f32)
                    m1 = jnp.zeros((16,), _f32)
                    for hd in range(heads):
                        a_sc = v[hd] * v[4 + hd]
                        ab = lax.broadcast_in_dim(a_sc, (16,), ())
                        m0 = m0 + ab * hrow[pl.ds(hd * GNN_H, 16)]
                        m1 = m1 + ab * hrow[pl.ds(hd * GNN_H + 16, 16)]
                    orow = orows[b].at[e]
                    orow[pl.ds(0, 16)] = m0
                    orow[pl.ds(16, 16)] = m1

        pltpu.sync_copy(src_hbm.at[pl.ds(base, w)], sidx[0])
        pltpu.sync_copy(dst_hbm.at[pl.ds(base, w)], didx[0])
        issue_gathers(0)
        issue_idx(1, 1)

        @pl.loop(0, nb // 2)
        def _(tt):
            for b in range(2):
                o = 1 - b
                wi = tt * 2 + b

                @pl.when(wi + 1 < nb)
                def _():
                    drain_idx(o)
                    issue_gathers(o)

                drain_gathers(b)

                @pl.when(wi >= 2)
                def _():
                    drain_scatter(b)

                for k in range(w // 16):
                    scidx[b][pl.ds(k * 16, 16)] = didx[b][pl.ds(k * 16, 16)]

                @pl.when(wi + 2 < nb)
                def _():
                    issue_idx(wi + 2, b)

                compute(b)
                pltpu.async_copy(orows[b], acc.at[scidx[b]], scsem[b],
                                 add=True)

        drain_scatter(0)
        drain_scatter(1)
        plsc.subcore_barrier()
        pltpu.sync_copy(acc.at[pl.ds(sid * PR, PR)],
                        out_hbm.at[cid, pl.ds(sid * PR, PR)])

    return kp


def _make_sc_pass_a4(w=W, depth=4):
    nb = EP_TILE // w
    ow = 16
    mesh = plsc.VectorSubcoreMesh(core_axis_name="c", subcore_axis_name="s")
    scratch = (
        [pltpu.VMEM((w,), jnp.int32)] * (3 * depth) +
        [pltpu.VMEM((w, 16), _f32)] * (2 * depth) +
        [pltpu.VMEM((w, ow), _f32)] * depth +
        [pltpu.VMEM((64, ow), _f32)] +
        [pltpu.VMEM_SHARED((NPAD, ow), _f32)] +
        [pltpu.SemaphoreType.DMA] * (3 * depth)
    )

    @functools.partial(
        pl.kernel, mesh=mesh,
        compiler_params=pltpu.CompilerParams(use_tc_tiling_on_sc=False),
        out_type=jax.ShapeDtypeStruct((2, NPAD, ow), _f32),
        scratch_types=scratch,
    )
    def ka(src_hbm, dst_hbm, tabs_hbm, tabd_hbm, out_hbm, *scr):
        d = depth
        sidx = scr[0:d]
        didx = scr[d:2 * d]
        scidx = scr[2 * d:3 * d]
        srows = scr[3 * d:4 * d]
        drows = scr[4 * d:5 * d]
        orows = scr[5 * d:6 * d]
        zbuf = scr[6 * d]
        acc = scr[6 * d + 1]
        gsem = scr[6 * d + 2:7 * d + 2]
        idxsem = scr[7 * d + 2:8 * d + 2]
        scsem = scr[8 * d + 2:9 * d + 2]

        cid = lax.axis_index("c")
        sid = lax.axis_index("s")
        wid = sid * 2 + cid
        base = wid * EP_TILE

        @pl.loop(0, 64)
        def _(i):
            zbuf[i, :] = jnp.zeros((16,), _f32)

        @pl.loop(0, PR // 64)
        def _(j):
            pltpu.sync_copy(zbuf, acc.at[pl.ds(sid * PR + j * 64, 64)])

        plsc.subcore_barrier()

        def issue_idx(wi, b):
            off = base + wi * w
            pltpu.async_copy(src_hbm.at[pl.ds(off, w)], sidx[b], idxsem[b])
            pltpu.async_copy(dst_hbm.at[pl.ds(off, w)], didx[b], idxsem[b])

        def drain_idx(b):
            pltpu.make_async_copy(src_hbm.at[pl.ds(0, w)], sidx[b],
                                  idxsem[b]).wait()
            pltpu.make_async_copy(dst_hbm.at[pl.ds(0, w)], didx[b],
                                  idxsem[b]).wait()

        def issue_gathers(b):
            pltpu.async_copy(tabs_hbm.at[sidx[b]], srows[b], gsem[b])
            pltpu.async_copy(tabd_hbm.at[didx[b]], drows[b], gsem[b])

        def drain_gathers(b):
            pltpu.make_async_copy(tabs_hbm.at[pl.ds(0, w)], srows[b],
                                  gsem[b]).wait()
            pltpu.make_async_copy(tabd_hbm.at[pl.ds(0, w)], drows[b],
                                  gsem[b]).wait()

        def drain_scatter(b):
            pltpu.make_async_copy(out_hbm.at[0, pl.ds(0, w)], orows[b],
                                  scsem[b]).wait()

        for p in range(2):
            pltpu.sync_copy(src_hbm.at[pl.ds(base + p * w, w)], sidx[p])
            pltpu.sync_copy(dst_hbm.at[pl.ds(base + p * w, w)], didx[p])
            issue_gathers(p)
        issue_idx(2, 2)

        @pl.loop(0, nb // depth)
        def _(tt):
            for b in range(depth):
                wi = tt * depth + b
                b2 = (b + 2) % depth
                b3 = (b + 3) % depth

                @pl.when(wi + 2 < nb)
                def _():
                    drain_idx(b2)
                    issue_gathers(b2)

                @pl.when(wi + 3 < nb)
                def _():
                    issue_idx(wi + 3, b3)

                drain_gathers(b)

                @pl.when(wi >= depth)
                def _():
                    drain_scatter(b)

                for k in range(w // 16):
                    scidx[b][pl.ds(k * 16, 16)] = didx[b][pl.ds(k * 16, 16)]

                @plsc.parallel_loop(0, w, unroll=4)
                def _(e):
                    t = srows[b].at[e][...] + drows[b].at[e][...]
                    t = jnp.where(t > 0, t, 0.2 * t)
                    orows[b].at[e][...] = jnp.exp(t)

                pltpu.async_copy(orows[b], acc.at[scidx[b]], scsem[b],
                                 add=True)

        for b in range(depth):
            drain_scatter(b)
        plsc.subcore_barrier()
        pltpu.sync_copy(acc.at[pl.ds(sid * PR, PR)],
                        out_hbm.at[cid, pl.ds(sid * PR, PR)])

    return ka


_sc_pass_a = _make_sc_pass_a4()
_sc_pass_b1 = _make_sc_pass("b", 128, HEADS1, w=64)
_sc_pass_b2 = _make_sc_pass("b", GNN_H, 1)


def kernel(x_static, hist_speed, time_idx, day_idx, edge_index, time_emb,
           day_emb, gru_W_ih, gru_W_hh, gru_b_ih, gru_b_hh, gat1_W,
           gat1_att_src, gat1_att_dst, gat1_bias, gat2_W, gat2_att_src,
           gat2_att_dst, gat2_bias):
    pad_n = NPAD - N
    xs = jnp.pad(x_static, ((0, pad_n), (0, 0)))
    hist = jnp.pad(hist_speed.reshape(N, HIST_T), ((0, pad_n), (0, 0)))
    tif = jnp.pad(time_idx.astype(jnp.int32).reshape(N, 1), ((0, pad_n), (0, 0)))
    dif = jnp.pad(day_idx.astype(jnp.int32).reshape(N, 1), ((0, pad_n), (0, 0)))
    demb = jnp.pad(day_emb, ((0, 1), (0, 0)))
    wih = gru_W_ih.reshape(1, 3 * RNN_H)
    whh = gru_W_hh.T
    bih = gru_b_ih.reshape(1, 3 * RNN_H)
    bhh = gru_b_hh.reshape(1, 3 * RNN_H)
    headmask = (jnp.arange(HEADS1 * GNN_H)[:, None] // GNN_H
                == jnp.arange(16)[None, :]).astype(_f32)
    a1s = headmask * gat1_att_src.reshape(HEADS1 * GNN_H, 1)
    a1d = headmask * gat1_att_dst.reshape(HEADS1 * GNN_H, 1)
    col0 = (jnp.arange(16)[None, :] == 0).astype(_f32)
    a2s = gat2_att_src.reshape(GNN_H, 1) * col0
    a2d = gat2_att_dst.reshape(GNN_H, 1) * col0
    b1 = gat1_bias.reshape(1, GNN_H)
    b2 = gat2_bias.reshape(1, GNN_H)

    pad_e = E_PAD - E
    fill = jnp.full((pad_e,), NPAD - 1, jnp.int32)
    src = jnp.concatenate([edge_index[0].astype(jnp.int32), fill])
    dst = jnp.concatenate([edge_index[1].astype(jnp.int32), fill])

    h1, asrc1, adst1 = _run_k1(xs, hist, tif, dif, time_emb, demb, wih,
                               whh, bih, bhh, gat1_W, a1s, a1d)
    dpart1 = _sc_pass_a(src, dst, asrc1, adst1)
    bdst1, selfm1 = _run_mid(HEADS1, 0.25, False, dpart1[0], dpart1[1],
                             asrc1, adst1, h1)
    opart1 = _sc_pass_b1(src, dst, asrc1, bdst1, h1)
    h2, asrc2, adst2 = _run_k3(opart1[0], opart1[1], selfm1, b1,
                               gat2_W, a2s, a2d)
    dpart2 = _sc_pass_a(src, dst, asrc2, adst2)
    bdst2, selfm2 = _run_mid(1, 1.0, False, dpart2[0], dpart2[1], asrc2, adst2, h2)
    opart2 = _sc_pass_b2(src, dst, asrc2, bdst2, h2)
    out = _run_k5(opart2[0], opart2[1], selfm2, b2)
    return out[:N]


# speedup vs baseline: 64.8815x; 1.0005x over previous
import functools

import jax
import jax.numpy as jnp
from jax import lax
from jax.experimental import pallas as pl
from jax.experimental.pallas import tpu as pltpu
from jax.experimental.pallas import tpu_sc as plsc

N = 50000
E = 800000
NPAD = 50176
R = 1024
HIST_T = 12
RNN_H = 64
GNN_H = 32
HEADS1 = 4
GNN_IN = 112

NW = 32
W = 128
E_PAD = 802816
EP_TILE = E_PAD // NW
NB = EP_TILE // W
PR = NPAD // 16

_f32 = jnp.float32


def _k1_body(xs_ref, hist_ref, tif_ref, dif_ref,
             temb_ref, demb_ref, wih_ref, whh_ref, bih_ref, bhh_ref,
             w1_ref, asm_ref, adm_ref,
             h1_ref, asrc_ref, adst_ref):
    rows = xs_ref.shape[0]
    t_oh = (lax.broadcasted_iota(jnp.int32, (rows, 288), 1) == tif_ref[:, :]).astype(_f32)
    d_oh = (lax.broadcasted_iota(jnp.int32, (rows, 8), 1) == dif_ref[:, :]).astype(_f32)
    t_emb = jnp.dot(t_oh, temb_ref[:, :], preferred_element_type=_f32)
    d_emb = jnp.dot(d_oh, demb_ref[:, :], preferred_element_type=_f32)
    wih = wih_ref[:, :]
    bih = bih_ref[:, :]
    bhh = bhh_ref[:, :]
    half = rows // 2
    hs = [jnp.zeros((half, RNN_H), dtype=_f32) for _ in range(2)]
    xs_t = [[hist_ref[p * half:(p + 1) * half, t:t + 1]
             for t in range(HIST_T)] for p in range(2)]
    for t in range(HIST_T):
        gis = [jnp.dot(xs_t[p][t], wih, preferred_element_type=_f32) + bih
               for p in range(2)]
        ghs = [jnp.dot(hs[p], whh_ref[:, :], preferred_element_type=_f32)
               + bhh for p in range(2)]
        for p in range(2):
            gi, gh, h = gis[p], ghs[p], hs[p]
            i_r = gi[:, 0:RNN_H]
            i_z = gi[:, RNN_H:2 * RNN_H]
            i_n = gi[:, 2 * RNN_H:3 * RNN_H]
            h_r = gh[:, 0:RNN_H]
            h_z = gh[:, RNN_H:2 * RNN_H]
            h_n = gh[:, 2 * RNN_H:3 * RNN_H]
            r = 0.5 * (1.0 + jnp.tanh(0.5 * (i_r + h_r)))
            z = 0.5 * (1.0 + jnp.tanh(0.5 * (i_z + h_z)))
            nn_ = jnp.tanh(i_n + r * h_n)
            hs[p] = (1.0 - z) * nn_ + z * h
    h = jnp.concatenate(hs, axis=0)
    feat = jnp.concatenate([xs_ref[:, :], t_emb, d_emb, h], axis=1)
    h1 = jnp.dot(feat, w1_ref[:, :], preferred_element_type=_f32)
    h1_ref[:, :] = h1
    asrc_ref[:, :] = jnp.dot(h1, asm_ref[:, :], preferred_element_type=_f32)
    adst_ref[:, :] = jnp.dot(h1, adm_ref[:, :], preferred_element_type=_f32)


def _run_k1(xs, hist, tif, dif, temb, demb, wih, whh, bih, bhh, w1, asm, adm):
    nblk = NPAD // R
    full = lambda shape: pl.BlockSpec(shape, lambda i: (0, 0))
    rowblk = lambda c: pl.BlockSpec((R, c), lambda i: (i, 0))
    return pl.pallas_call(
        _k1_body,
        grid=(nblk,),
        in_specs=[rowblk(16), rowblk(HIST_T), rowblk(1), rowblk(1),
                  full((288, 16)), full((8, 16)), full((1, 192)),
                  full((RNN_H, 192)), full((1, 192)), full((1, 192)),
                  full((GNN_IN, 128)), full((128, 16)), full((128, 16))],
        out_specs=[rowblk(128), rowblk(16), rowblk(16)],
        out_shape=[jax.ShapeDtypeStruct((NPAD, 128), _f32),
                   jax.ShapeDtypeStruct((NPAD, 16), _f32),
                   jax.ShapeDtypeStruct((NPAD, 16), _f32)],
    )(xs, hist, tif, dif, temb, demb, wih, whh, bih, bhh, w1, asm, adm)


def _mid_body(heads, scale, sigma, d0_ref, d1_ref, asrc_ref, adst_ref, h_ref,
              bdst_ref, selfm_ref):
    rows = d0_ref.shape[0]
    a_s = asrc_ref[:, 0:4]
    a_d = adst_ref[:, 0:4]
    t = a_s + a_d
    ex_self = jnp.exp(jnp.where(t > 0, t, 0.2 * t))
    denom = d0_ref[:, 0:4] + d1_ref[:, 0:4] + ex_self
    rdenom = scale / (denom + 1e-16)
    zs8 = jnp.zeros((rows, 8), dtype=_f32)
    bdst_ref[:, :] = jnp.concatenate([a_d, rdenom, zs8], axis=1)
    alpha_self = ex_self * rdenom
    if sigma:
        ev = jnp.zeros((rows, GNN_H // 2), dtype=_f32)
        od = jnp.zeros((rows, GNN_H // 2), dtype=_f32)
        for hd in range(heads):
            a = alpha_self[:, hd:hd + 1]
            ev = ev + a * h_ref[:, hd * GNN_H:hd * GNN_H + GNN_H:2]
            od = od + a * h_ref[:, hd * GNN_H + 1:hd * GNN_H + GNN_H:2]
        selfm_ref[:, :] = jnp.concatenate([ev, od], axis=1)
    else:
        acc = jnp.zeros((rows, GNN_H), dtype=_f32)
        for hd in range(heads):
            acc = acc + alpha_self[:, hd:hd + 1] * h_ref[:, hd * GNN_H:(hd + 1) * GNN_H]
        selfm_ref[:, :] = acc


def _run_mid(heads, scale, sigma, d0, d1, asrc, adst, h):
    nblk = NPAD // R
    hw = h.shape[1]
    rowblk = lambda c: pl.BlockSpec((R, c), lambda i: (i, 0))
    return pl.pallas_call(
        functools.partial(_mid_body, heads, scale, sigma),
        grid=(nblk,),
        in_specs=[rowblk(16), rowblk(16), rowblk(16), rowblk(16), rowblk(hw)],
        out_specs=[rowblk(16), rowblk(GNN_H)],
        out_shape=[jax.ShapeDtypeStruct((NPAD, 16), _f32),
                   jax.ShapeDtypeStruct((NPAD, GNN_H), _f32)],
    )(d0, d1, asrc, adst, h)


def _k3_body(p0_ref, p1_ref, selfm_ref, b1_ref, w2_ref, a2s_ref, a2d_ref,
             h2_ref, asrc_ref, adst_ref):
    rows = p0_ref.shape[0]
    x = p0_ref[:, :] + p1_ref[:, :] + selfm_ref[:, :] + b1_ref[:, :]
    x = jnp.maximum(x, 0.0)
    h2 = jnp.dot(x, w2_ref[:, :], preferred_element_type=_f32)
    h2_ref[:, :] = h2
    asrc_ref[:, :] = jnp.dot(h2, a2s_ref[:, :], preferred_element_type=_f32)
    adst_ref[:, :] = jnp.dot(h2, a2d_ref[:, :], preferred_element_type=_f32)


def _run_k3(p0, p1, selfm, b1, w2, a2s, a2d):
    nblk = NPAD // R
    full = lambda shape: pl.BlockSpec(shape, lambda i: (0, 0))
    rowblk = lambda c: pl.BlockSpec((R, c), lambda i: (i, 0))
    return pl.pallas_call(
        _k3_body,
        grid=(nblk,),
        in_specs=[rowblk(GNN_H), rowblk(GNN_H), rowblk(GNN_H),
                  full((1, GNN_H)), full((GNN_H, GNN_H)),
                  full((GNN_H, 16)), full((GNN_H, 16))],
        out_specs=[rowblk(GNN_H), rowblk(16), rowblk(16)],
        out_shape=[jax.ShapeDtypeStruct((NPAD, GNN_H), _f32),
                   jax.ShapeDtypeStruct((NPAD, 16), _f32),
                   jax.ShapeDtypeStruct((NPAD, 16), _f32)],
    )(p0, p1, selfm, b1, w2, a2s, a2d)


def _k5_body(p0_ref, p1_ref, selfm_ref, b2_ref, out_ref):
    x = p0_ref[:, :] + p1_ref[:, :] + selfm_ref[:, :] + b2_ref[:, :]
    out_ref[:, :] = jnp.maximum(x, 0.0)


def _run_k5(p0, p1, selfm, b2):
    nblk = NPAD // R
    full = lambda shape: pl.BlockSpec(shape, lambda i: (0, 0))
    rowblk = lambda c: pl.BlockSpec((R, c), lambda i: (i, 0))
    return pl.pallas_call(
        _k5_body,
        grid=(nblk,),
        in_specs=[rowblk(GNN_H), rowblk(GNN_H), rowblk(GNN_H), full((1, GNN_H))],
        out_specs=rowblk(GNN_H),
        out_shape=jax.ShapeDtypeStruct((NPAD, GNN_H), _f32),
    )(p0, p1, selfm, b2)


def _make_sc_pass(mode, hw=0, heads=0, w=W):
    nb = EP_TILE // w
    ow = 16 if mode == "a" else GNN_H
    mesh = plsc.VectorSubcoreMesh(core_axis_name="c", subcore_axis_name="s")
    scratch = (
        [pltpu.VMEM((w,), jnp.int32)] * 6 +
        [pltpu.VMEM((w, 16), _f32)] * 4 +
        ([pltpu.VMEM((w, hw), _f32)] * 2 if mode == "b" else []) +
        [pltpu.VMEM((w, ow), _f32)] * 2 +
        [pltpu.VMEM((64, ow), _f32)] +
        [pltpu.VMEM_SHARED((NPAD, ow), _f32)] +
        [pltpu.SemaphoreType.DMA] * 6
    )

    @functools.partial(
        pl.kernel, mesh=mesh,
        compiler_params=pltpu.CompilerParams(use_tc_tiling_on_sc=False),
        out_type=jax.ShapeDtypeStruct((2, NPAD, ow), _f32),
        scratch_types=scratch,
    )
    def kp(*refs):
        if mode == "a":
            (src_hbm, dst_hbm, tabs_hbm, tabd_hbm, out_hbm,
             sidx0, didx0, scidx0, sidx1, didx1, scidx1,
             srows0, drows0, srows1, drows1,
             orows0, orows1, zbuf, acc,
             gsem0, gsem1, idxsem0, idxsem1, scsem0, scsem1) = refs
            hrows = None
        else:
            (src_hbm, dst_hbm, tabs_hbm, tabd_hbm, h_hbm, out_hbm,
             sidx0, didx0, scidx0, sidx1, didx1, scidx1,
             srows0, drows0, srows1, drows1, hrows0, hrows1,
             orows0, orows1, zbuf, acc,
             gsem0, gsem1, idxsem0, idxsem1, scsem0, scsem1) = refs
            hrows = (hrows0, hrows1)
        sidx = (sidx0, sidx1)
        didx = (didx0, didx1)
        scidx = (scidx0, scidx1)
        srows = (srows0, srows1)
        drows = (drows0, drows1)
        orows = (orows0, orows1)
        gsem = (gsem0, gsem1)
        idxsem = (idxsem0, idxsem1)
        scsem = (scsem0, scsem1)

        cid = lax.axis_index("c")
        sid = lax.axis_index("s")
        wid = sid * 2 + cid
        base = wid * EP_TILE
        lane = lax.iota(jnp.int32, 16)
        exmask = lane < 4

        @pl.loop(0, 64)
        def _(i):
            for c in range(ow // 16):
                zbuf[i, pl.ds(c * 16, 16)] = jnp.zeros((16,), _f32)

        @pl.loop(0, PR // 64)
        def _(j):
            pltpu.sync_copy(zbuf, acc.at[pl.ds(sid * PR + j * 64, 64)])

        plsc.subcore_barrier()

        def issue_idx(wi, b):
            off = base + wi * w
            pltpu.async_copy(src_hbm.at[pl.ds(off, w)], sidx[b], idxsem[b])
            pltpu.async_copy(dst_hbm.at[pl.ds(off, w)], didx[b], idxsem[b])

        def drain_idx(b):
            pltpu.make_async_copy(src_hbm.at[pl.ds(0, w)], sidx[b],
                                  idxsem[b]).wait()
            pltpu.make_async_copy(dst_hbm.at[pl.ds(0, w)], didx[b],
                                  idxsem[b]).wait()

        def issue_gathers(b):
            pltpu.async_copy(tabs_hbm.at[sidx[b]], srows[b], gsem[b])
            pltpu.async_copy(tabd_hbm.at[didx[b]], drows[b], gsem[b])
            if mode == "b":
                pltpu.async_copy(h_hbm.at[sidx[b]], hrows[b], gsem[b])

        def drain_gathers(b):
            pltpu.make_async_copy(tabs_hbm.at[pl.ds(0, w)], srows[b],
                                  gsem[b]).wait()
            pltpu.make_async_copy(tabd_hbm.at[pl.ds(0, w)], drows[b],
                                  gsem[b]).wait()
            if mode == "b":
                pltpu.make_async_copy(h_hbm.at[pl.ds(0, w)], hrows[b],
                                      gsem[b]).wait()

        def drain_scatter(b):
            pltpu.make_async_copy(out_hbm.at[0, pl.ds(0, w)], orows[b],
                                  scsem[b]).wait()

        def compute(b):
            @plsc.parallel_loop(0, w, unroll=4)
            def _(e):
                t = srows[b].at[e][...] + drows[b].at[e][...]
                t = jnp.where(t > 0, t, 0.2 * t)
                if mode == "a":
                    orows[b].at[e][...] = jnp.exp(t)
                else:
                    v = jnp.where(exmask, jnp.exp(t), t)
                    hrow = hrows[b].at[e]
                    m0 = jnp.zeros((16,), _f32)
                    m1 = jnp.zeros((16,), _f32)
                    for hd in range(heads):
                        a_sc = v[hd] * v[4 + hd]
                        ab = lax.broadcast_in_dim(a_sc, (16,), ())
                        m0 = m0 + ab * hrow[pl.ds(hd * GNN_H, 16)]
                        m1 = m1 + ab * hrow[pl.ds(hd * GNN_H + 16, 16)]
                    orow = orows[b].at[e]
                    orow[pl.ds(0, 16)] = m0
                    orow[pl.ds(16, 16)] = m1

        pltpu.sync_copy(src_hbm.at[pl.ds(base, w)], sidx[0])
        pltpu.sync_copy(dst_hbm.at[pl.ds(base, w)], didx[0])
        issue_gathers(0)
        issue_idx(1, 1)

        @pl.loop(0, nb // 2)
        def _(tt):
            for b in range(2):
                o = 1 - b
                wi = tt * 2 + b

                @pl.when(wi + 1 < nb)
                def _():
                    drain_idx(o)
                    issue_gathers(o)

                drain_gathers(b)

                @pl.when(wi >= 2)
                def _():
                    drain_scatter(b)

                for k in range(w // 16):
                    scidx[b][pl.ds(k * 16, 16)] = didx[b][pl.ds(k * 16, 16)]

                @pl.when(wi + 2 < nb)
                def _():
                    issue_idx(wi + 2, b)

                compute(b)
                pltpu.async_copy(orows[b], acc.at[scidx[b]], scsem[b],
                                 add=True)

        drain_scatter(0)
        drain_scatter(1)
        plsc.subcore_barrier()
        pltpu.sync_copy(acc.at[pl.ds(sid * PR, PR)],
                        out_hbm.at[cid, pl.ds(sid * PR, PR)])

    return kp


def _make_sc_pass_a4(w=W, depth=4):
    nb = EP_TILE // w
    ow = 16
    mesh = plsc.VectorSubcoreMesh(core_axis_name="c", subcore_axis_name="s")
    scratch = (
        [pltpu.VMEM((w,), jnp.int32)] * (3 * depth) +
        [pltpu.VMEM((w, 16), _f32)] * (2 * depth) +
        [pltpu.VMEM((w, ow), _f32)] * depth +
        [pltpu.VMEM((64, ow), _f32)] +
        [pltpu.VMEM_SHARED((NPAD, ow), _f32)] +
        [pltpu.SemaphoreType.DMA] * (3 * depth)
    )

    @functools.partial(
        pl.kernel, mesh=mesh,
        compiler_params=pltpu.CompilerParams(use_tc_tiling_on_sc=False),
        out_type=jax.ShapeDtypeStruct((2, NPAD, ow), _f32),
        scratch_types=scratch,
    )
    def ka(src_hbm, dst_hbm, tabs_hbm, tabd_hbm, out_hbm, *scr):
        d = depth
        sidx = scr[0:d]
        didx = scr[d:2 * d]
        scidx = scr[2 * d:3 * d]
        srows = scr[3 * d:4 * d]
        drows = scr[4 * d:5 * d]
        orows = scr[5 * d:6 * d]
        zbuf = scr[6 * d]
        acc = scr[6 * d + 1]
        gsem = scr[6 * d + 2:7 * d + 2]
        idxsem = scr[7 * d + 2:8 * d + 2]
        scsem = scr[8 * d + 2:9 * d + 2]

        cid = lax.axis_index("c")
        sid = lax.axis_index("s")
        wid = sid * 2 + cid
        base = wid * EP_TILE

        @pl.loop(0, 64)
        def _(i):
            zbuf[i, :] = jnp.zeros((16,), _f32)

        @pl.loop(0, PR // 64)
        def _(j):
            pltpu.sync_copy(zbuf, acc.at[pl.ds(sid * PR + j * 64, 64)])

        plsc.subcore_barrier()

        def issue_idx(wi, b):
            off = base + wi * w
            pltpu.async_copy(src_hbm.at[pl.ds(off, w)], sidx[b], idxsem[b])
            pltpu.async_copy(dst_hbm.at[pl.ds(off, w)], didx[b], idxsem[b])

        def drain_idx(b):
            pltpu.make_async_copy(src_hbm.at[pl.ds(0, w)], sidx[b],
                                  idxsem[b]).wait()
            pltpu.make_async_copy(dst_hbm.at[pl.ds(0, w)], didx[b],
                                  idxsem[b]).wait()

        def issue_gathers(b):
            pltpu.async_copy(tabs_hbm.at[sidx[b]], srows[b], gsem[b])
            pltpu.async_copy(tabd_hbm.at[didx[b]], drows[b], gsem[b])

        def drain_gathers(b):
            pltpu.make_async_copy(tabs_hbm.at[pl.ds(0, w)], srows[b],
                                  gsem[b]).wait()
            pltpu.make_async_copy(tabd_hbm.at[pl.ds(0, w)], drows[b],
                                  gsem[b]).wait()

        def drain_scatter(b):
            pltpu.make_async_copy(out_hbm.at[0, pl.ds(0, w)], orows[b],
                                  scsem[b]).wait()

        for p in range(2):
            pltpu.sync_copy(src_hbm.at[pl.ds(base + p * w, w)], sidx[p])
            pltpu.sync_copy(dst_hbm.at[pl.ds(base + p * w, w)], didx[p])
            issue_gathers(p)
        issue_idx(2, 2)

        @pl.loop(0, nb // depth)
        def _(tt):
            for b in range(depth):
                wi = tt * depth + b
                b2 = (b + 2) % depth
                b3 = (b + 3) % depth

                @pl.when(wi + 2 < nb)
                def _():
                    drain_idx(b2)
                    issue_gathers(b2)

                @pl.when(wi + 3 < nb)
                def _():
                    issue_idx(wi + 3, b3)

                drain_gathers(b)

                @pl.when(wi >= depth)
                def _():
                    drain_scatter(b)

                for k in range(w // 16):
                    scidx[b][pl.ds(k * 16, 16)] = didx[b][pl.ds(k * 16, 16)]

                @plsc.parallel_loop(0, w, unroll=4)
                def _(e):
                    t = srows[b].at[e][...] + drows[b].at[e][...]
                    t = jnp.where(t > 0, t, 0.2 * t)
                    orows[b].at[e][...] = jnp.exp(t)

                pltpu.async_copy(orows[b], acc.at[scidx[b]], scsem[b],
                                 add=True)

        for b in range(depth):
            drain_scatter(b)
        plsc.subcore_barrier()
        pltpu.sync_copy(acc.at[pl.ds(sid * PR, PR)],
                        out_hbm.at[cid, pl.ds(sid * PR, PR)])

    return ka


_sc_pass_a = _make_sc_pass_a4()
_sc_pass_b1 = _make_sc_pass("b", 128, HEADS1, w=64)
_sc_pass_b2 = _make_sc_pass("b", GNN_H, 1)


def kernel(x_static, hist_speed, time_idx, day_idx, edge_index, time_emb,
           day_emb, gru_W_ih, gru_W_hh, gru_b_ih, gru_b_hh, gat1_W,
           gat1_att_src, gat1_att_dst, gat1_bias, gat2_W, gat2_att_src,
           gat2_att_dst, gat2_bias):
    pad_n = NPAD - N
    xs = jnp.pad(x_static, ((0, pad_n), (0, 0)))
    hist = jnp.pad(hist_speed.reshape(N, HIST_T), ((0, pad_n), (0, 0)))
    tif = jnp.pad(time_idx.astype(jnp.int32).reshape(N, 1), ((0, pad_n), (0, 0)))
    dif = jnp.pad(day_idx.astype(jnp.int32).reshape(N, 1), ((0, pad_n), (0, 0)))
    demb = jnp.pad(day_emb, ((0, 1), (0, 0)))
    wih = gru_W_ih.reshape(1, 3 * RNN_H)
    whh = gru_W_hh.T
    bih = gru_b_ih.reshape(1, 3 * RNN_H)
    bhh = gru_b_hh.reshape(1, 3 * RNN_H)
    headmask = (jnp.arange(HEADS1 * GNN_H)[:, None] // GNN_H
                == jnp.arange(16)[None, :]).astype(_f32)
    a1s = headmask * gat1_att_src.reshape(HEADS1 * GNN_H, 1)
    a1d = headmask * gat1_att_dst.reshape(HEADS1 * GNN_H, 1)
    col0 = (jnp.arange(16)[None, :] == 0).astype(_f32)
    a2s = gat2_att_src.reshape(GNN_H, 1) * col0
    a2d = gat2_att_dst.reshape(GNN_H, 1) * col0
    b1 = gat1_bias.reshape(1, GNN_H)
    b2 = gat2_bias.reshape(1, GNN_H)

    pad_e = E_PAD - E
    fill = jnp.full((pad_e,), NPAD - 1, jnp.int32)
    src = jnp.concatenate([edge_index[0].astype(jnp.int32), fill])
    dst = jnp.concatenate([edge_index[1].astype(jnp.int32), fill])

    h1, asrc1, adst1 = _run_k1(xs, hist, tif, dif, time_emb, demb, wih,
                               whh, bih, bhh, gat1_W, a1s, a1d)
    dpart1 = _sc_pass_a(src, dst, asrc1, adst1)
    bdst1, selfm1 = _run_mid(HEADS1, 0.25, False, dpart1[0], dpart1[1],
                             asrc1, adst1, h1)
    opart1 = _sc_pass_b1(src, dst, asrc1, bdst1, h1)
    h2, asrc2, adst2 = _run_k3(opart1[0], opart1[1], selfm1, b1,
                               gat2_W, a2s, a2d)
    dpart2 = _sc_pass_a(src, dst, asrc2, adst2)
    bdst2, selfm2 = _run_mid(1, 1.0, False, dpart2[0], dpart2[1], asrc2, adst2, h2)
    opart2 = _sc_pass_b2(src, dst, asrc2, bdst2, h2)
    out = _run_k5(opart2[0], opart2[1], selfm2, b2)
    return out[:N]
